# baseline (speedup 1.0000x reference)
import functools

import jax
import jax.numpy as jnp
from jax import lax
from jax.experimental import pallas as pl
from jax.experimental.pallas import tpu as pltpu
from jax.experimental.pallas import tpu_sc as plsc

N = 10000
E = 320000
IN_DIM = 128
HID = 32
NA = 10240
CHUNK = 128
NW = 32
KCH = 80
EP = NW * KCH * CHUNK
W1D = 80
ROWS_PER_TILE = NA // 16


def _relu(v):
    return jnp.maximum(v, 0.0)


def _dot(a, w):
    return jnp.dot(a, w, preferred_element_type=jnp.float32)


def _tc1_body(x_ref, w_ref, b_ref, h0_ref, h0e_ref):
    h0 = _dot(x_ref[...], w_ref[...]) + b_ref[...]
    h0_ref[...] = h0
    br = h0.shape[0]
    ones = jnp.ones((br, 1), jnp.float32)
    pad = jnp.zeros((br, W1D - 2 * HID - 1), jnp.float32)
    h0e_ref[...] = jnp.concatenate([h0, h0 * h0, ones, pad], axis=1)


def _tc1(x_pad, w, b, block=512):
    grid = (NA // block,)
    return pl.pallas_call(
        _tc1_body,
        grid=grid,
        in_specs=[
            pl.BlockSpec((block, IN_DIM), lambda i: (i, 0)),
            pl.BlockSpec((IN_DIM, HID), lambda i: (0, 0)),
            pl.BlockSpec((1, HID), lambda i: (0, 0)),
        ],
        out_specs=[
            pl.BlockSpec((block, HID), lambda i: (i, 0)),
            pl.BlockSpec((block, W1D), lambda i: (i, 0)),
        ],
        out_shape=[
            jax.ShapeDtypeStruct((NA, HID), jnp.float32),
            jax.ShapeDtypeStruct((NA, W1D), jnp.float32),
        ],
    )(x_pad, w, b)


def _make_sc_segsum(width):
    mesh = plsc.VectorSubcoreMesh(core_axis_name="c", subcore_axis_name="s")

    @functools.partial(
        pl.kernel,
        out_type=jax.ShapeDtypeStruct((2, NA, width), jnp.float32),
        mesh=mesh,
        compiler_params=pltpu.CompilerParams(use_tc_tiling_on_sc=False),
        scratch_types=[
            pltpu.VMEM((KCH, CHUNK), jnp.int32),
            pltpu.VMEM((KCH, CHUNK), jnp.int32),
            pltpu.VMEM((CHUNK, width), jnp.float32),
            pltpu.VMEM((CHUNK, width), jnp.float32),
            pltpu.VMEM_SHARED((NA, width), jnp.float32),
            pltpu.SemaphoreType.DMA,
            pltpu.SemaphoreType.DMA,
        ],
    )
    def sc_kernel(rows_hbm, srcm, dstm, zeros_hbm, out_hbm,
                  src_v, dst_v, buf0, buf1, acc, sem0, sem1):
        cid = lax.axis_index("c")
        sid = lax.axis_index("s")
        w = cid * 16 + sid
        pltpu.sync_copy(zeros_hbm.at[pl.ds(sid * ROWS_PER_TILE, ROWS_PER_TILE)],
                        acc.at[pl.ds(sid * ROWS_PER_TILE, ROWS_PER_TILE)])
        pltpu.sync_copy(srcm.at[w], src_v)
        pltpu.sync_copy(dstm.at[w], dst_v)
        plsc.subcore_barrier()

        pltpu.async_copy(rows_hbm.at[src_v.at[0]], buf0, sem0)

        def body2(k, carry):
            j0 = 2 * k
            j1 = j0 + 1
            j2 = jnp.minimum(j0 + 2, KCH - 1)
            pltpu.make_async_copy(rows_hbm.at[src_v.at[j0]], buf0, sem0).wait()
            pltpu.async_copy(rows_hbm.at[src_v.at[j1]], buf1, sem1)
            pltpu.sync_copy(buf0, acc.at[dst_v.at[j0]], add=True)
            pltpu.make_async_copy(rows_hbm.at[src_v.at[j1]], buf1, sem1).wait()
            pltpu.async_copy(rows_hbm.at[src_v.at[j2]], buf0, sem0)
            pltpu.sync_copy(buf1, acc.at[dst_v.at[j1]], add=True)
            return carry

        lax.fori_loop(0, KCH // 2, body2, 0)
        pltpu.make_async_copy(rows_hbm.at[src_v.at[0]], buf0, sem0).wait()
        plsc.subcore_barrier()
        pltpu.sync_copy(acc.at[pl.ds(sid * ROWS_PER_TILE, ROWS_PER_TILE)],
                        out_hbm.at[cid, pl.ds(sid * ROWS_PER_TILE, ROWS_PER_TILE)])

    return sc_kernel


def _tc2_body(h0_ref, p1_ref, w1, b1, w2, b2, h1_ref, stats_ref):
    st = p1_ref[0] + p1_ref[1]
    stats_ref[...] = st
    pre = h0_ref[...] + st[:, :HID]
    t = _relu(_dot(pre, w1[...]) + b1[...])
    h1_ref[...] = _relu(_dot(t, w2[...]) + b2[...])


def _tc2(h0, part1, w1, b1, w2, b2, block=512):
    grid = (NA // block,)
    wspec = pl.BlockSpec((HID, HID), lambda i: (0, 0))
    bspec = pl.BlockSpec((1, HID), lambda i: (0, 0))
    return pl.pallas_call(
        _tc2_body,
        grid=grid,
        in_specs=[
            pl.BlockSpec((block, HID), lambda i: (i, 0)),
            pl.BlockSpec((2, block, W1D), lambda i: (0, i, 0)),
            wspec, bspec, wspec, bspec,
        ],
        out_specs=[
            pl.BlockSpec((block, HID), lambda i: (i, 0)),
            pl.BlockSpec((block, W1D), lambda i: (i, 0)),
        ],
        out_shape=[
            jax.ShapeDtypeStruct((NA, HID), jnp.float32),
            jax.ShapeDtypeStruct((NA, W1D), jnp.float32),
        ],
    )(h0, part1, w1, b1, w2, b2)


def _tc3_body(h0_ref, h1_ref, stats_ref, p2_ref, z_ref, *refs):
    (g1w1, g1b1, g1w2, g1b2,
     dw1, db1, dw2, db2, dw3, db3, dw4, db4,
     slw, slb, srw, srb, stw, stb,
     fw1, fb1, fw2, fb2, fw3, fb3,
     mw1, mb1, mw2, mb2, mw3, mb3,
     gw1, gb1, gw2, gb2, gw3, gb3,
     nw1, nb1, nw2, nb2, nw3, nb3, nw4, nb4,
     l1_ref, scal_ref, h0p_ref) = refs

    h0 = h0_ref[...]
    h1 = h1_ref[...]
    st = stats_ref[...]
    s3 = p2_ref[0] + p2_ref[1]

    pre = h1 + s3
    t = _relu(_dot(pre, g1w1[...]) + g1b1[...])
    l1 = _dot(t, g1w2[...]) + g1b2[...]
    l1_ref[...] = l1

    t = _relu(_dot(l1, dw1[...]) + db1[...])
    t = _relu(_dot(t, dw2[...]) + db2[...])
    t = _relu(_dot(t, dw3[...]) + db3[...])
    deg = _relu(_dot(t, dw4[...]) + db4[...])

    s1 = st[:, :HID]
    s2 = st[:, HID:2 * HID]
    cnt = st[:, 2 * HID:2 * HID + 1]
    denom = jnp.maximum(cnt, 1.0)
    mean_n = s1 / denom
    mean_sq = s2 / denom
    std_raw = jnp.sqrt(_relu(mean_sq - mean_n * mean_n) + 1e-5)
    mean_neigh = (_dot(mean_n, slw[...]) + slb[...]
                  + _dot(h0, srw[...]) + srb[...])
    s = _dot(std_raw, stw[...]) + stb[...]

    t = _relu(_dot(l1, fw1[...]) + fb1[...])
    t = _relu(_dot(t, fw2[...]) + fb2[...])
    h0p_ref[...] = _dot(t, fw3[...]) + fb3[...]

    t = _relu(_dot(l1, mw1[...]) + mb1[...])
    t = _relu(_dot(t, mw2[...]) + mb2[...])
    g_mean = _dot(t, mw3[...]) + mb3[...]
    t = _relu(_dot(l1, gw1[...]) + gb1[...])
    t = _relu(_dot(t, gw2[...]) + gb2[...])
    g_sigma = _dot(t, gw3[...]) + gb3[...]
    escale = jnp.exp(g_sigma)

    def gen(var):
        t = _relu(_dot(var, nw1[...]) + nb1[...])
        t = _relu(_dot(t, nw2[...]) + nb2[...])
        t = _relu(_dot(t, nw3[...]) + nb3[...])
        return _dot(t, nw4[...]) + nb4[...]

    n0 = gen(g_mean + escale * z_ref[0])
    n1 = gen(g_mean + escale * z_ref[1])
    gen_mean = 0.5 * (n0 + n1)
    u = 0.5 * jnp.abs(n0 - n1)

    ss = jnp.sum(s * s, axis=1, keepdims=True)
    uu = jnp.sum(u * u, axis=1, keepdims=True)
    us = jnp.sum(u * s, axis=1, keepdims=True)
    det_t = 1.0 + ss
    det_g = 1.0 + uu
    trace = HID + ss - (uu + us * us) / det_g
    d = gen_mean - mean_neigh
    zq = (jnp.sum(d * d, axis=1, keepdims=True)
          - jnp.sum(u * d, axis=1, keepdims=True) ** 2 / det_g)
    kl = 0.5 * (jnp.log(det_g / det_t) - HID + trace + zq)
    br = deg.shape[0]
    scal_ref[...] = jnp.concatenate(
        [deg, det_t, det_g, trace, zq, kl, jnp.zeros((br, 2), jnp.float32)],
        axis=1)


def _tc3(h0, h1, stats, part2, zpad, wlist, block=512):
    grid = (NA // block,)

    def fullspec(a):
        nd = a.ndim
        return pl.BlockSpec(a.shape, lambda i, _nd=nd: (0,) * _nd)

    in_specs = [
        pl.BlockSpec((block, HID), lambda i: (i, 0)),
        pl.BlockSpec((block, HID), lambda i: (i, 0)),
        pl.BlockSpec((block, W1D), lambda i: (i, 0)),
        pl.BlockSpec((2, block, HID), lambda i: (0, i, 0)),
        pl.BlockSpec((2, block, HID), lambda i: (0, i, 0)),
    ] + [fullspec(a) for a in wlist]
    return pl.pallas_call(
        _tc3_body,
        grid=grid,
        in_specs=in_specs,
        out_specs=[
            pl.BlockSpec((block, HID), lambda i: (i, 0)),
            pl.BlockSpec((block, 8), lambda i: (i, 0)),
            pl.BlockSpec((block, IN_DIM), lambda i: (i, 0)),
        ],
        out_shape=[
            jax.ShapeDtypeStruct((NA, HID), jnp.float32),
            jax.ShapeDtypeStruct((NA, 8), jnp.float32),
            jax.ShapeDtypeStruct((NA, IN_DIM), jnp.float32),
        ],
    )(h0, h1, stats, part2, zpad, *wlist)


def _b2(b):
    return b.reshape(1, -1)


def kernel(x, edge_index, params):
    src = edge_index[0].astype(jnp.int32)
    dst = edge_index[1].astype(jnp.int32)
    srcm = jnp.concatenate([src, jnp.zeros((EP - E,), jnp.int32)]).reshape(NW, KCH, CHUNK)
    dstm = jnp.concatenate([dst, jnp.full((EP - E,), NA - 1, jnp.int32)]).reshape(NW, KCH, CHUNK)
    x_pad = jnp.concatenate([x, jnp.zeros((NA - N, IN_DIM), jnp.float32)])
    zeros80 = jnp.zeros((NA, W1D), jnp.float32)
    zeros32 = jnp.zeros((NA, HID), jnp.float32)

    p = params
    lw, lb = p["lin"]
    h0, h0e = _tc1(x_pad, lw, _b2(lb))

    part1 = _make_sc_segsum(W1D)(h0e, srcm, dstm, zeros80)

    (g0w1, g0b1), (g0w2, g0b2) = p["gin"][0]
    h1, stats = _tc2(h0, part1, g0w1, _b2(g0b1), g0w2, _b2(g0b2))

    part2 = _make_sc_segsum(HID)(h1, srcm, dstm, zeros32)

    z = jax.random.normal(jax.random.fold_in(jax.random.key(1), 0),
                          (2, N, HID), jnp.float32)
    zpad = jnp.concatenate([z, jnp.zeros((2, NA - N, HID), jnp.float32)], axis=1)

    wlist = []
    for (w, b) in p["gin"][1]:
        wlist += [w, _b2(b)]
    for (w, b) in p["deg"]:
        wlist += [w, _b2(b)]
    for key in ("sage_l", "sage_r", "std_lin"):
        w, b = p[key]
        wlist += [w, _b2(b)]
    for key in ("feat", "mlp_mean", "mlp_sigma", "gen"):
        for (w, b) in p[key]:
            wlist += [w, _b2(b)]

    l1, scal, h0p = _tc3(h0, h1, stats, part2, zpad, wlist)

    return (h0[:N], l1[:N], scal[:N, 0:1], (h0p[:N],),
            ((scal[:N, 1], scal[:N, 2], scal[:N, 3], scal[:N, 4], scal[:N, 5]),))


# speedup vs baseline: 1.1616x; 1.1616x over previous
import functools

import jax
import jax.numpy as jnp
from jax import lax
from jax.experimental import pallas as pl
from jax.experimental.pallas import tpu as pltpu
from jax.experimental.pallas import tpu_sc as plsc

N = 10000
E = 320000
IN_DIM = 128
HID = 32
NA = 10240
CHUNK = 128
NW = 32
KCH = 80
EP = NW * KCH * CHUNK
ROW1 = 2 * HID
DEGW = 8
ROWS_PER_TILE = NA // 16


def _relu(v):
    return jnp.maximum(v, 0.0)


def _dot(a, w):
    return jnp.dot(a, w, preferred_element_type=jnp.float32)


def _tc1_body(x_ref, w_ref, b_ref, h0_ref, h0e_ref):
    h0 = _dot(x_ref[...], w_ref[...]) + b_ref[...]
    h0_ref[...] = h0
    h0e_ref[...] = jnp.concatenate([h0, h0 * h0], axis=1)


def _tc1(x_pad, w, b, block=512):
    grid = (NA // block,)
    return pl.pallas_call(
        _tc1_body,
        grid=grid,
        in_specs=[
            pl.BlockSpec((block, IN_DIM), lambda i: (i, 0)),
            pl.BlockSpec((IN_DIM, HID), lambda i: (0, 0)),
            pl.BlockSpec((1, HID), lambda i: (0, 0)),
        ],
        out_specs=[
            pl.BlockSpec((block, HID), lambda i: (i, 0)),
            pl.BlockSpec((block, ROW1), lambda i: (i, 0)),
        ],
        out_shape=[
            jax.ShapeDtypeStruct((NA, HID), jnp.float32),
            jax.ShapeDtypeStruct((NA, ROW1), jnp.float32),
        ],
    )(x_pad, w, b)


def _make_sc_segsum_deg(width):
    mesh = plsc.VectorSubcoreMesh(core_axis_name="c", subcore_axis_name="s")

    @functools.partial(
        pl.kernel,
        out_type=[
            jax.ShapeDtypeStruct((2, NA, width), jnp.float32),
            jax.ShapeDtypeStruct((2, NA, DEGW), jnp.float32),
        ],
        mesh=mesh,
        compiler_params=pltpu.CompilerParams(use_tc_tiling_on_sc=False),
        scratch_types=[
            pltpu.VMEM((KCH, CHUNK), jnp.int32),
            pltpu.VMEM((KCH, CHUNK), jnp.int32),
            pltpu.VMEM((CHUNK, width), jnp.float32),
            pltpu.VMEM((CHUNK, width), jnp.float32),
            pltpu.VMEM((CHUNK, DEGW), jnp.float32),
            pltpu.VMEM_SHARED((NA, width), jnp.float32),
            pltpu.VMEM_SHARED((NA, DEGW), jnp.float32),
            pltpu.SemaphoreType.DMA,
            pltpu.SemaphoreType.DMA,
        ],
    )
    def sc_kernel(rows_hbm, srcm, dstm, zeros_hbm, zerosd_hbm, ones_hbm,
                  out_hbm, outd_hbm,
                  src_v, dst_v, buf0, buf1, ones_v, acc, accd, sem0, sem1):
        cid = lax.axis_index("c")
        sid = lax.axis_index("s")
        w = cid * 16 + sid
        pltpu.sync_copy(zeros_hbm.at[pl.ds(sid * ROWS_PER_TILE, ROWS_PER_TILE)],
                        acc.at[pl.ds(sid * ROWS_PER_TILE, ROWS_PER_TILE)])
        pltpu.sync_copy(zerosd_hbm.at[pl.ds(sid * ROWS_PER_TILE, ROWS_PER_TILE)],
                        accd.at[pl.ds(sid * ROWS_PER_TILE, ROWS_PER_TILE)])
        pltpu.sync_copy(srcm.at[w], src_v)
        pltpu.sync_copy(dstm.at[w], dst_v)
        pltpu.sync_copy(ones_hbm, ones_v)
        plsc.subcore_barrier()

        pltpu.async_copy(rows_hbm.at[src_v.at[0]], buf0, sem0)

        def body2(k, carry):
            j0 = 2 * k
            j1 = j0 + 1
            j2 = jnp.minimum(j0 + 2, KCH - 1)
            pltpu.make_async_copy(rows_hbm.at[src_v.at[j0]], buf0, sem0).wait()
            pltpu.async_copy(rows_hbm.at[src_v.at[j1]], buf1, sem1)
            pltpu.sync_copy(buf0, acc.at[dst_v.at[j0]], add=True)
            pltpu.sync_copy(ones_v, accd.at[dst_v.at[j0]], add=True)
            pltpu.make_async_copy(rows_hbm.at[src_v.at[j1]], buf1, sem1).wait()
            pltpu.async_copy(rows_hbm.at[src_v.at[j2]], buf0, sem0)
            pltpu.sync_copy(buf1, acc.at[dst_v.at[j1]], add=True)
            pltpu.sync_copy(ones_v, accd.at[dst_v.at[j1]], add=True)
            return carry

        lax.fori_loop(0, KCH // 2, body2, 0)
        pltpu.make_async_copy(rows_hbm.at[src_v.at[0]], buf0, sem0).wait()
        plsc.subcore_barrier()
        pltpu.sync_copy(acc.at[pl.ds(sid * ROWS_PER_TILE, ROWS_PER_TILE)],
                        out_hbm.at[cid, pl.ds(sid * ROWS_PER_TILE, ROWS_PER_TILE)])
        pltpu.sync_copy(accd.at[pl.ds(sid * ROWS_PER_TILE, ROWS_PER_TILE)],
                        outd_hbm.at[cid, pl.ds(sid * ROWS_PER_TILE, ROWS_PER_TILE)])

    return sc_kernel


def _make_sc_segsum(width):
    mesh = plsc.VectorSubcoreMesh(core_axis_name="c", subcore_axis_name="s")

    @functools.partial(
        pl.kernel,
        out_type=jax.ShapeDtypeStruct((2, NA, width), jnp.float32),
        mesh=mesh,
        compiler_params=pltpu.CompilerParams(use_tc_tiling_on_sc=False),
        scratch_types=[
            pltpu.VMEM((KCH, CHUNK), jnp.int32),
            pltpu.VMEM((KCH, CHUNK), jnp.int32),
            pltpu.VMEM((CHUNK, width), jnp.float32),
            pltpu.VMEM((CHUNK, width), jnp.float32),
            pltpu.VMEM_SHARED((NA, width), jnp.float32),
            pltpu.SemaphoreType.DMA,
            pltpu.SemaphoreType.DMA,
        ],
    )
    def sc_kernel(rows_hbm, srcm, dstm, zeros_hbm, out_hbm,
                  src_v, dst_v, buf0, buf1, acc, sem0, sem1):
        cid = lax.axis_index("c")
        sid = lax.axis_index("s")
        w = cid * 16 + sid
        pltpu.sync_copy(zeros_hbm.at[pl.ds(sid * ROWS_PER_TILE, ROWS_PER_TILE)],
                        acc.at[pl.ds(sid * ROWS_PER_TILE, ROWS_PER_TILE)])
        pltpu.sync_copy(srcm.at[w], src_v)
        pltpu.sync_copy(dstm.at[w], dst_v)
        plsc.subcore_barrier()

        pltpu.async_copy(rows_hbm.at[src_v.at[0]], buf0, sem0)

        def body2(k, carry):
            j0 = 2 * k
            j1 = j0 + 1
            j2 = jnp.minimum(j0 + 2, KCH - 1)
            pltpu.make_async_copy(rows_hbm.at[src_v.at[j0]], buf0, sem0).wait()
            pltpu.async_copy(rows_hbm.at[src_v.at[j1]], buf1, sem1)
            pltpu.sync_copy(buf0, acc.at[dst_v.at[j0]], add=True)
            pltpu.make_async_copy(rows_hbm.at[src_v.at[j1]], buf1, sem1).wait()
            pltpu.async_copy(rows_hbm.at[src_v.at[j2]], buf0, sem0)
            pltpu.sync_copy(buf1, acc.at[dst_v.at[j1]], add=True)
            return carry

        lax.fori_loop(0, KCH // 2, body2, 0)
        pltpu.make_async_copy(rows_hbm.at[src_v.at[0]], buf0, sem0).wait()
        plsc.subcore_barrier()
        pltpu.sync_copy(acc.at[pl.ds(sid * ROWS_PER_TILE, ROWS_PER_TILE)],
                        out_hbm.at[cid, pl.ds(sid * ROWS_PER_TILE, ROWS_PER_TILE)])

    return sc_kernel


def _tc2_body(h0_ref, p1_ref, pd_ref, w1, b1, w2, b2, h1_ref, st_ref, cnt_ref):
    st = p1_ref[0] + p1_ref[1]
    st_ref[...] = st
    cnt_ref[...] = pd_ref[0] + pd_ref[1]
    pre = h0_ref[...] + st[:, :HID]
    t = _relu(_dot(pre, w1[...]) + b1[...])
    h1_ref[...] = _relu(_dot(t, w2[...]) + b2[...])


def _tc2(h0, part1, partd, w1, b1, w2, b2, block=512):
    grid = (NA // block,)
    wspec = pl.BlockSpec((HID, HID), lambda i: (0, 0))
    bspec = pl.BlockSpec((1, HID), lambda i: (0, 0))
    return pl.pallas_call(
        _tc2_body,
        grid=grid,
        in_specs=[
            pl.BlockSpec((block, HID), lambda i: (i, 0)),
            pl.BlockSpec((2, block, ROW1), lambda i: (0, i, 0)),
            pl.BlockSpec((2, block, DEGW), lambda i: (0, i, 0)),
            wspec, bspec, wspec, bspec,
        ],
        out_specs=[
            pl.BlockSpec((block, HID), lambda i: (i, 0)),
            pl.BlockSpec((block, ROW1), lambda i: (i, 0)),
            pl.BlockSpec((block, DEGW), lambda i: (i, 0)),
        ],
        out_shape=[
            jax.ShapeDtypeStruct((NA, HID), jnp.float32),
            jax.ShapeDtypeStruct((NA, ROW1), jnp.float32),
            jax.ShapeDtypeStruct((NA, DEGW), jnp.float32),
        ],
    )(h0, part1, partd, w1, b1, w2, b2)


def _tc3_body(h0_ref, h1_ref, st_ref, cnt_ref, p2_ref, z_ref, *refs):
    (g1w1, g1b1, g1w2, g1b2,
     dw1, db1, dw2, db2, dw3, db3, dw4, db4,
     slw, slb, srw, srb, stw, stb,
     fw1, fb1, fw2, fb2, fw3, fb3,
     mw1, mb1, mw2, mb2, mw3, mb3,
     gw1, gb1, gw2, gb2, gw3, gb3,
     nw1, nb1, nw2, nb2, nw3, nb3, nw4, nb4,
     l1_ref, scal_ref, h0p_ref) = refs

    h0 = h0_ref[...]
    h1 = h1_ref[...]
    st = st_ref[...]
    s3 = p2_ref[0] + p2_ref[1]

    pre = h1 + s3
    t = _relu(_dot(pre, g1w1[...]) + g1b1[...])
    l1 = _dot(t, g1w2[...]) + g1b2[...]
    l1_ref[...] = l1

    t = _relu(_dot(l1, dw1[...]) + db1[...])
    t = _relu(_dot(t, dw2[...]) + db2[...])
    t = _relu(_dot(t, dw3[...]) + db3[...])
    deg = _relu(_dot(t, dw4[...]) + db4[...])

    s1 = st[:, :HID]
    s2 = st[:, HID:2 * HID]
    cnt = cnt_ref[...][:, 0:1]
    denom = jnp.maximum(cnt, 1.0)
    mean_n = s1 / denom
    mean_sq = s2 / denom
    std_raw = jnp.sqrt(_relu(mean_sq - mean_n * mean_n) + 1e-5)
    mean_neigh = (_dot(mean_n, slw[...]) + slb[...]
                  + _dot(h0, srw[...]) + srb[...])
    s = _dot(std_raw, stw[...]) + stb[...]

    t = _relu(_dot(l1, fw1[...]) + fb1[...])
    t = _relu(_dot(t, fw2[...]) + fb2[...])
    h0p_ref[...] = _dot(t, fw3[...]) + fb3[...]

    t = _relu(_dot(l1, mw1[...]) + mb1[...])
    t = _relu(_dot(t, mw2[...]) + mb2[...])
    g_mean = _dot(t, mw3[...]) + mb3[...]
    t = _relu(_dot(l1, gw1[...]) + gb1[...])
    t = _relu(_dot(t, gw2[...]) + gb2[...])
    g_sigma = _dot(t, gw3[...]) + gb3[...]
    escale = jnp.exp(g_sigma)

    def gen(var):
        t = _relu(_dot(var, nw1[...]) + nb1[...])
        t = _relu(_dot(t, nw2[...]) + nb2[...])
        t = _relu(_dot(t, nw3[...]) + nb3[...])
        return _dot(t, nw4[...]) + nb4[...]

    n0 = gen(g_mean + escale * z_ref[0])
    n1 = gen(g_mean + escale * z_ref[1])
    gen_mean = 0.5 * (n0 + n1)
    u = 0.5 * jnp.abs(n0 - n1)

    ss = jnp.sum(s * s, axis=1, keepdims=True)
    uu = jnp.sum(u * u, axis=1, keepdims=True)
    us = jnp.sum(u * s, axis=1, keepdims=True)
    det_t = 1.0 + ss
    det_g = 1.0 + uu
    trace = HID + ss - (uu + us * us) / det_g
    d = gen_mean - mean_neigh
    zq = (jnp.sum(d * d, axis=1, keepdims=True)
          - jnp.sum(u * d, axis=1, keepdims=True) ** 2 / det_g)
    kl = 0.5 * (jnp.log(det_g / det_t) - HID + trace + zq)
    br = deg.shape[0]
    scal_ref[...] = jnp.concatenate(
        [deg, det_t, det_g, trace, zq, kl, jnp.zeros((br, 2), jnp.float32)],
        axis=1)


def _tc3(h0, h1, st, cnt, part2, zpad, wlist, block=512):
    grid = (NA // block,)

    def fullspec(a):
        nd = a.ndim
        return pl.BlockSpec(a.shape, lambda i, _nd=nd: (0,) * _nd)

    in_specs = [
        pl.BlockSpec((block, HID), lambda i: (i, 0)),
        pl.BlockSpec((block, HID), lambda i: (i, 0)),
        pl.BlockSpec((block, ROW1), lambda i: (i, 0)),
        pl.BlockSpec((block, DEGW), lambda i: (i, 0)),
        pl.BlockSpec((2, block, HID), lambda i: (0, i, 0)),
        pl.BlockSpec((2, block, HID), lambda i: (0, i, 0)),
    ] + [fullspec(a) for a in wlist]
    return pl.pallas_call(
        _tc3_body,
        grid=grid,
        in_specs=in_specs,
        out_specs=[
            pl.BlockSpec((block, HID), lambda i: (i, 0)),
            pl.BlockSpec((block, 8), lambda i: (i, 0)),
            pl.BlockSpec((block, IN_DIM), lambda i: (i, 0)),
        ],
        out_shape=[
            jax.ShapeDtypeStruct((NA, HID), jnp.float32),
            jax.ShapeDtypeStruct((NA, 8), jnp.float32),
            jax.ShapeDtypeStruct((NA, IN_DIM), jnp.float32),
        ],
    )(h0, h1, st, cnt, part2, zpad, *wlist)


def _b2(b):
    return b.reshape(1, -1)


def kernel(x, edge_index, params):
    src = edge_index[0].astype(jnp.int32)
    dst = edge_index[1].astype(jnp.int32)
    pad_dst = N + (jnp.arange(EP - E, dtype=jnp.int32) % (NA - N))
    srcm = jnp.concatenate([src, jnp.zeros((EP - E,), jnp.int32)]).reshape(NW, KCH, CHUNK)
    dstm = jnp.concatenate([dst, pad_dst]).reshape(NW, KCH, CHUNK)
    x_pad = jnp.concatenate([x, jnp.zeros((NA - N, IN_DIM), jnp.float32)])
    zeros64 = jnp.zeros((NA, ROW1), jnp.float32)
    zeros32 = jnp.zeros((NA, HID), jnp.float32)
    zeros8 = jnp.zeros((NA, DEGW), jnp.float32)
    ones8 = jnp.ones((CHUNK, DEGW), jnp.float32)

    p = params
    lw, lb = p["lin"]
    h0, h0e = _tc1(x_pad, lw, _b2(lb))

    part1, partd = _make_sc_segsum_deg(ROW1)(h0e, srcm, dstm, zeros64, zeros8, ones8)

    (g0w1, g0b1), (g0w2, g0b2) = p["gin"][0]
    h1, st, cnt = _tc2(h0, part1, partd, g0w1, _b2(g0b1), g0w2, _b2(g0b2))

    part2 = _make_sc_segsum(HID)(h1, srcm, dstm, zeros32)

    z = jax.random.normal(jax.random.fold_in(jax.random.key(1), 0),
                          (2, N, HID), jnp.float32)
    zpad = jnp.concatenate([z, jnp.zeros((2, NA - N, HID), jnp.float32)], axis=1)

    wlist = []
    for (w, b) in p["gin"][1]:
        wlist += [w, _b2(b)]
    for (w, b) in p["deg"]:
        wlist += [w, _b2(b)]
    for key in ("sage_l", "sage_r", "std_lin"):
        w, b = p[key]
        wlist += [w, _b2(b)]
    for key in ("feat", "mlp_mean", "mlp_sigma", "gen"):
        for (w, b) in p[key]:
            wlist += [w, _b2(b)]

    l1, scal, h0p = _tc3(h0, h1, st, cnt, part2, zpad, wlist)

    return (h0[:N], l1[:N], scal[:N, 0:1], (h0p[:N],),
            ((scal[:N, 1], scal[:N, 2], scal[:N, 3], scal[:N, 4], scal[:N, 5]),))


# speedup vs baseline: 1.6211x; 1.3956x over previous
import functools

import jax
import jax.numpy as jnp
from jax import lax
from jax.experimental import pallas as pl
from jax.experimental.pallas import tpu as pltpu
from jax.experimental.pallas import tpu_sc as plsc

N = 10000
E = 320000
IN_DIM = 128
HID = 32
NA = 10240
CHUNK = 128
NW = 32
KCH = 80
SLOT = KCH * CHUNK
EP = NW * SLOT
REAL_PW = E // NW
PAD_PW = SLOT - REAL_PW
ROW1 = 2 * HID
DEGW = 8
ROWS_PER_TILE = NA // 16
BLK = 400

_off = jnp.arange(EP, dtype=jnp.int32) % SLOT
_wk = jnp.arange(EP, dtype=jnp.int32) // SLOT
_IS_REAL = _off < REAL_PW
_REAL_IDX = jnp.minimum(_wk * REAL_PW + _off, E - 1)
_p = _wk * PAD_PW + jnp.maximum(_off - REAL_PW, 0)
_PAD_SRC = _p % N
_PAD_DST = N + _p % (NA - N)

_Z = jax.random.normal(jax.random.fold_in(jax.random.key(1), 0),
                       (2, N, HID), jnp.float32)
_ZEROS_ROW1 = jnp.zeros((NA, ROW1), jnp.float32)
_ZEROS_HID = jnp.zeros((NA, HID), jnp.float32)
_ZEROS_DEG = jnp.zeros((NA, DEGW), jnp.float32)
_ONES_DEG = jnp.ones((CHUNK, DEGW), jnp.float32)


def _relu(v):
    return jnp.maximum(v, 0.0)


def _dot(a, w):
    return jnp.dot(a, w, preferred_element_type=jnp.float32)


def _tc1_body(x_ref, w_ref, b_ref, h0_ref, h0e_ref):
    h0 = _dot(x_ref[...], w_ref[...]) + b_ref[...]
    h0_ref[...] = h0
    h0e_ref[...] = jnp.concatenate([h0, h0 * h0], axis=1)


def _tc1(x, w, b):
    grid = (N // BLK,)
    return pl.pallas_call(
        _tc1_body,
        grid=grid,
        in_specs=[
            pl.BlockSpec((BLK, IN_DIM), lambda i: (i, 0)),
            pl.BlockSpec((IN_DIM, HID), lambda i: (0, 0)),
            pl.BlockSpec((1, HID), lambda i: (0, 0)),
        ],
        out_specs=[
            pl.BlockSpec((BLK, HID), lambda i: (i, 0)),
            pl.BlockSpec((BLK, ROW1), lambda i: (i, 0)),
        ],
        out_shape=[
            jax.ShapeDtypeStruct((N, HID), jnp.float32),
            jax.ShapeDtypeStruct((N, ROW1), jnp.float32),
        ],
    )(x, w, b)


def _make_sc_segsum_deg(width):
    mesh = plsc.VectorSubcoreMesh(core_axis_name="c", subcore_axis_name="s")

    @functools.partial(
        pl.kernel,
        out_type=[
            jax.ShapeDtypeStruct((2, NA, width), jnp.float32),
            jax.ShapeDtypeStruct((2, NA, DEGW), jnp.float32),
        ],
        mesh=mesh,
        compiler_params=pltpu.CompilerParams(use_tc_tiling_on_sc=False),
        scratch_types=[
            pltpu.VMEM((KCH, CHUNK), jnp.int32),
            pltpu.VMEM((KCH, CHUNK), jnp.int32),
            pltpu.VMEM((CHUNK, width), jnp.float32),
            pltpu.VMEM((CHUNK, width), jnp.float32),
            pltpu.VMEM((CHUNK, DEGW), jnp.float32),
            pltpu.VMEM_SHARED((NA, width), jnp.float32),
            pltpu.VMEM_SHARED((NA, DEGW), jnp.float32),
            pltpu.SemaphoreType.DMA,
            pltpu.SemaphoreType.DMA,
        ],
    )
    def sc_kernel(rows_hbm, srcm, dstm, zeros_hbm, zerosd_hbm, ones_hbm,
                  out_hbm, outd_hbm,
                  src_v, dst_v, buf0, buf1, ones_v, acc, accd, sem0, sem1):
        cid = lax.axis_index("c")
        sid = lax.axis_index("s")
        w = cid * 16 + sid
        pltpu.sync_copy(zeros_hbm.at[pl.ds(sid * ROWS_PER_TILE, ROWS_PER_TILE)],
                        acc.at[pl.ds(sid * ROWS_PER_TILE, ROWS_PER_TILE)])
        pltpu.sync_copy(zerosd_hbm.at[pl.ds(sid * ROWS_PER_TILE, ROWS_PER_TILE)],
                        accd.at[pl.ds(sid * ROWS_PER_TILE, ROWS_PER_TILE)])
        pltpu.sync_copy(srcm.at[w], src_v)
        pltpu.sync_copy(dstm.at[w], dst_v)
        pltpu.sync_copy(ones_hbm, ones_v)
        plsc.subcore_barrier()

        pltpu.async_copy(rows_hbm.at[src_v.at[0]], buf0, sem0)

        def body2(k, carry):
            j0 = 2 * k
            j1 = j0 + 1
            j2 = jnp.minimum(j0 + 2, KCH - 1)
            pltpu.make_async_copy(rows_hbm.at[src_v.at[j0]], buf0, sem0).wait()
            pltpu.async_copy(rows_hbm.at[src_v.at[j1]], buf1, sem1)
            pltpu.sync_copy(buf0, acc.at[dst_v.at[j0]], add=True)
            pltpu.sync_copy(ones_v, accd.at[dst_v.at[j0]], add=True)
            pltpu.make_async_copy(rows_hbm.at[src_v.at[j1]], buf1, sem1).wait()
            pltpu.async_copy(rows_hbm.at[src_v.at[j2]], buf0, sem0)
            pltpu.sync_copy(buf1, acc.at[dst_v.at[j1]], add=True)
            pltpu.sync_copy(ones_v, accd.at[dst_v.at[j1]], add=True)
            return carry

        lax.fori_loop(0, KCH // 2, body2, 0)
        pltpu.make_async_copy(rows_hbm.at[src_v.at[0]], buf0, sem0).wait()
        plsc.subcore_barrier()
        pltpu.sync_copy(acc.at[pl.ds(sid * ROWS_PER_TILE, ROWS_PER_TILE)],
                        out_hbm.at[cid, pl.ds(sid * ROWS_PER_TILE, ROWS_PER_TILE)])
        pltpu.sync_copy(accd.at[pl.ds(sid * ROWS_PER_TILE, ROWS_PER_TILE)],
                        outd_hbm.at[cid, pl.ds(sid * ROWS_PER_TILE, ROWS_PER_TILE)])

    return sc_kernel


def _make_sc_segsum(width):
    mesh = plsc.VectorSubcoreMesh(core_axis_name="c", subcore_axis_name="s")

    @functools.partial(
        pl.kernel,
        out_type=jax.ShapeDtypeStruct((2, NA, width), jnp.float32),
        mesh=mesh,
        compiler_params=pltpu.CompilerParams(use_tc_tiling_on_sc=False),
        scratch_types=[
            pltpu.VMEM((KCH, CHUNK), jnp.int32),
            pltpu.VMEM((KCH, CHUNK), jnp.int32),
            pltpu.VMEM((CHUNK, width), jnp.float32),
            pltpu.VMEM((CHUNK, width), jnp.float32),
            pltpu.VMEM_SHARED((NA, width), jnp.float32),
            pltpu.SemaphoreType.DMA,
            pltpu.SemaphoreType.DMA,
        ],
    )
    def sc_kernel(rows_hbm, srcm, dstm, zeros_hbm, out_hbm,
                  src_v, dst_v, buf0, buf1, acc, sem0, sem1):
        cid = lax.axis_index("c")
        sid = lax.axis_index("s")
        w = cid * 16 + sid
        pltpu.sync_copy(zeros_hbm.at[pl.ds(sid * ROWS_PER_TILE, ROWS_PER_TILE)],
                        acc.at[pl.ds(sid * ROWS_PER_TILE, ROWS_PER_TILE)])
        pltpu.sync_copy(srcm.at[w], src_v)
        pltpu.sync_copy(dstm.at[w], dst_v)
        plsc.subcore_barrier()

        pltpu.async_copy(rows_hbm.at[src_v.at[0]], buf0, sem0)

        def body2(k, carry):
            j0 = 2 * k
            j1 = j0 + 1
            j2 = jnp.minimum(j0 + 2, KCH - 1)
            pltpu.make_async_copy(rows_hbm.at[src_v.at[j0]], buf0, sem0).wait()
            pltpu.async_copy(rows_hbm.at[src_v.at[j1]], buf1, sem1)
            pltpu.sync_copy(buf0, acc.at[dst_v.at[j0]], add=True)
            pltpu.make_async_copy(rows_hbm.at[src_v.at[j1]], buf1, sem1).wait()
            pltpu.async_copy(rows_hbm.at[src_v.at[j2]], buf0, sem0)
            pltpu.sync_copy(buf1, acc.at[dst_v.at[j1]], add=True)
            return carry

        lax.fori_loop(0, KCH // 2, body2, 0)
        pltpu.make_async_copy(rows_hbm.at[src_v.at[0]], buf0, sem0).wait()
        plsc.subcore_barrier()
        pltpu.sync_copy(acc.at[pl.ds(sid * ROWS_PER_TILE, ROWS_PER_TILE)],
                        out_hbm.at[cid, pl.ds(sid * ROWS_PER_TILE, ROWS_PER_TILE)])

    return sc_kernel


def _tc2_body(h0_ref, p1_ref, pd_ref, w1, b1, w2, b2, h1_ref, st_ref, cnt_ref):
    st = p1_ref[0] + p1_ref[1]
    st_ref[...] = st
    cnt_ref[...] = pd_ref[0] + pd_ref[1]
    pre = h0_ref[...] + st[:, :HID]
    t = _relu(_dot(pre, w1[...]) + b1[...])
    h1_ref[...] = _relu(_dot(t, w2[...]) + b2[...])


def _tc2(h0, part1, partd, w1, b1, w2, b2):
    grid = (N // BLK,)
    wspec = pl.BlockSpec((HID, HID), lambda i: (0, 0))
    bspec = pl.BlockSpec((1, HID), lambda i: (0, 0))
    return pl.pallas_call(
        _tc2_body,
        grid=grid,
        in_specs=[
            pl.BlockSpec((BLK, HID), lambda i: (i, 0)),
            pl.BlockSpec((2, BLK, ROW1), lambda i: (0, i, 0)),
            pl.BlockSpec((2, BLK, DEGW), lambda i: (0, i, 0)),
            wspec, bspec, wspec, bspec,
        ],
        out_specs=[
            pl.BlockSpec((BLK, HID), lambda i: (i, 0)),
            pl.BlockSpec((BLK, ROW1), lambda i: (i, 0)),
            pl.BlockSpec((BLK, DEGW), lambda i: (i, 0)),
        ],
        out_shape=[
            jax.ShapeDtypeStruct((N, HID), jnp.float32),
            jax.ShapeDtypeStruct((N, ROW1), jnp.float32),
            jax.ShapeDtypeStruct((N, DEGW), jnp.float32),
        ],
    )(h0, part1, partd, w1, b1, w2, b2)


def _tc3_body(h0_ref, h1_ref, st_ref, cnt_ref, p2_ref, z_ref, *refs):
    (g1w1, g1b1, g1w2, g1b2,
     dw1, db1, dw2, db2, dw3, db3, dw4, db4,
     slw, slb, srw, srb, stw, stb,
     fw1, fb1, fw2, fb2, fw3, fb3,
     mw1, mb1, mw2, mb2, mw3, mb3,
     gw1, gb1, gw2, gb2, gw3, gb3,
     nw1, nb1, nw2, nb2, nw3, nb3, nw4, nb4,
     l1_ref, scal_ref, h0p_ref) = refs

    h0 = h0_ref[...]
    h1 = h1_ref[...]
    st = st_ref[...]
    s3 = p2_ref[0] + p2_ref[1]

    pre = h1 + s3
    t = _relu(_dot(pre, g1w1[...]) + g1b1[...])
    l1 = _dot(t, g1w2[...]) + g1b2[...]
    l1_ref[...] = l1

    t = _relu(_dot(l1, dw1[...]) + db1[...])
    t = _relu(_dot(t, dw2[...]) + db2[...])
    t = _relu(_dot(t, dw3[...]) + db3[...])
    deg = _relu(_dot(t, dw4[...]) + db4[...])

    s1 = st[:, :HID]
    s2 = st[:, HID:2 * HID]
    cnt = cnt_ref[...][:, 0:1]
    denom = jnp.maximum(cnt, 1.0)
    mean_n = s1 / denom
    mean_sq = s2 / denom
    std_raw = jnp.sqrt(_relu(mean_sq - mean_n * mean_n) + 1e-5)
    mean_neigh = (_dot(mean_n, slw[...]) + slb[...]
                  + _dot(h0, srw[...]) + srb[...])
    s = _dot(std_raw, stw[...]) + stb[...]

    t = _relu(_dot(l1, fw1[...]) + fb1[...])
    t = _relu(_dot(t, fw2[...]) + fb2[...])
    h0p_ref[...] = _dot(t, fw3[...]) + fb3[...]

    t = _relu(_dot(l1, mw1[...]) + mb1[...])
    t = _relu(_dot(t, mw2[...]) + mb2[...])
    g_mean = _dot(t, mw3[...]) + mb3[...]
    t = _relu(_dot(l1, gw1[...]) + gb1[...])
    t = _relu(_dot(t, gw2[...]) + gb2[...])
    g_sigma = _dot(t, gw3[...]) + gb3[...]
    escale = jnp.exp(g_sigma)

    def gen(var):
        t = _relu(_dot(var, nw1[...]) + nb1[...])
        t = _relu(_dot(t, nw2[...]) + nb2[...])
        t = _relu(_dot(t, nw3[...]) + nb3[...])
        return _dot(t, nw4[...]) + nb4[...]

    n0 = gen(g_mean + escale * z_ref[0])
    n1 = gen(g_mean + escale * z_ref[1])
    gen_mean = 0.5 * (n0 + n1)
    u = 0.5 * jnp.abs(n0 - n1)

    ss = jnp.sum(s * s, axis=1, keepdims=True)
    uu = jnp.sum(u * u, axis=1, keepdims=True)
    us = jnp.sum(u * s, axis=1, keepdims=True)
    det_t = 1.0 + ss
    det_g = 1.0 + uu
    trace = HID + ss - (uu + us * us) / det_g
    d = gen_mean - mean_neigh
    zq = (jnp.sum(d * d, axis=1, keepdims=True)
          - jnp.sum(u * d, axis=1, keepdims=True) ** 2 / det_g)
    kl = 0.5 * (jnp.log(det_g / det_t) - HID + trace + zq)
    br = deg.shape[0]
    scal_ref[...] = jnp.concatenate(
        [deg, det_t, det_g, trace, zq, kl, jnp.zeros((br, 2), jnp.float32)],
        axis=1)


def _tc3(h0, h1, st, cnt, part2, z, wlist):
    grid = (N // BLK,)

    def fullspec(a):
        nd = a.ndim
        return pl.BlockSpec(a.shape, lambda i, _nd=nd: (0,) * _nd)

    in_specs = [
        pl.BlockSpec((BLK, HID), lambda i: (i, 0)),
        pl.BlockSpec((BLK, HID), lambda i: (i, 0)),
        pl.BlockSpec((BLK, ROW1), lambda i: (i, 0)),
        pl.BlockSpec((BLK, DEGW), lambda i: (i, 0)),
        pl.BlockSpec((2, BLK, HID), lambda i: (0, i, 0)),
        pl.BlockSpec((2, BLK, HID), lambda i: (0, i, 0)),
    ] + [fullspec(a) for a in wlist]
    return pl.pallas_call(
        _tc3_body,
        grid=grid,
        in_specs=in_specs,
        out_specs=[
            pl.BlockSpec((BLK, HID), lambda i: (i, 0)),
            pl.BlockSpec((BLK, 8), lambda i: (i, 0)),
            pl.BlockSpec((BLK, IN_DIM), lambda i: (i, 0)),
        ],
        out_shape=[
            jax.ShapeDtypeStruct((N, HID), jnp.float32),
            jax.ShapeDtypeStruct((N, 8), jnp.float32),
            jax.ShapeDtypeStruct((N, IN_DIM), jnp.float32),
        ],
    )(h0, h1, st, cnt, part2, z, *wlist)


def _b2(b):
    return b.reshape(1, -1)


def kernel(x, edge_index, params):
    src = edge_index[0].astype(jnp.int32)
    dst = edge_index[1].astype(jnp.int32)
    srcm = jnp.where(_IS_REAL, src[_REAL_IDX], _PAD_SRC).reshape(NW, KCH, CHUNK)
    dstm = jnp.where(_IS_REAL, dst[_REAL_IDX], _PAD_DST).reshape(NW, KCH, CHUNK)

    p = params
    lw, lb = p["lin"]
    h0, h0e = _tc1(x, lw, _b2(lb))

    part1, partd = _make_sc_segsum_deg(ROW1)(
        h0e, srcm, dstm, _ZEROS_ROW1, _ZEROS_DEG, _ONES_DEG)

    (g0w1, g0b1), (g0w2, g0b2) = p["gin"][0]
    h1, st, cnt = _tc2(h0, part1, partd, g0w1, _b2(g0b1), g0w2, _b2(g0b2))

    part2 = _make_sc_segsum(HID)(h1, srcm, dstm, _ZEROS_HID)

    wlist = []
    for (w, b) in p["gin"][1]:
        wlist += [w, _b2(b)]
    for (w, b) in p["deg"]:
        wlist += [w, _b2(b)]
    for key in ("sage_l", "sage_r", "std_lin"):
        w, b = p[key]
        wlist += [w, _b2(b)]
    for key in ("feat", "mlp_mean", "mlp_sigma", "gen"):
        for (w, b) in p[key]:
            wlist += [w, _b2(b)]

    l1, scal, h0p = _tc3(h0, h1, st, cnt, part2, _Z, wlist)

    return (h0, l1, scal[:, 0:1], (h0p,),
            ((scal[:, 1], scal[:, 2], scal[:, 3], scal[:, 4], scal[:, 5]),))


# speedup vs baseline: 1.9359x; 1.1942x over previous
import functools

import jax
import jax.numpy as jnp
from jax import lax
from jax.experimental import pallas as pl
from jax.experimental.pallas import tpu as pltpu
from jax.experimental.pallas import tpu_sc as plsc

N = 10000
E = 320000
IN_DIM = 128
HID = 32
NA = 10240
CHUNK = 128
NW = 32
KCH = 80
SLOT = KCH * CHUNK
EP = NW * SLOT
REAL_PW = E // NW
PAD_PW = SLOT - REAL_PW
ROW1 = 2 * HID
DEGW = 8
ROWS_PER_TILE = NA // 16
BLK = 400

_p = jnp.arange(EP - E, dtype=jnp.int32)
_PAD_SRC_TAIL = _p % N
_PAD_DST_TAIL = N + _p % (NA - N)

_Z = jax.random.normal(jax.random.fold_in(jax.random.key(1), 0),
                       (2, N, HID), jnp.float32)
_ZEROS_ROW1 = jnp.zeros((NA, ROW1), jnp.float32)
_ZEROS_HID = jnp.zeros((NA, HID), jnp.float32)
_ZEROS_DEG = jnp.zeros((NA, DEGW), jnp.float32)
_ONES_DEG = jnp.ones((CHUNK, DEGW), jnp.float32)


def _relu(v):
    return jnp.maximum(v, 0.0)


def _dot(a, w):
    return jnp.dot(a, w, preferred_element_type=jnp.float32)


def _tc1_body(x_ref, w_ref, b_ref, h0_ref, h0e_ref):
    h0 = _dot(x_ref[...], w_ref[...]) + b_ref[...]
    h0_ref[...] = h0
    h0e_ref[...] = jnp.concatenate([h0, h0 * h0], axis=1)


def _tc1(x, w, b):
    grid = (N // BLK,)
    return pl.pallas_call(
        _tc1_body,
        grid=grid,
        in_specs=[
            pl.BlockSpec((BLK, IN_DIM), lambda i: (i, 0)),
            pl.BlockSpec((IN_DIM, HID), lambda i: (0, 0)),
            pl.BlockSpec((1, HID), lambda i: (0, 0)),
        ],
        out_specs=[
            pl.BlockSpec((BLK, HID), lambda i: (i, 0)),
            pl.BlockSpec((BLK, ROW1), lambda i: (i, 0)),
        ],
        out_shape=[
            jax.ShapeDtypeStruct((N, HID), jnp.float32),
            jax.ShapeDtypeStruct((N, ROW1), jnp.float32),
        ],
    )(x, w, b)


def _make_sc_segsum_deg(width):
    mesh = plsc.VectorSubcoreMesh(core_axis_name="c", subcore_axis_name="s")

    @functools.partial(
        pl.kernel,
        out_type=[
            jax.ShapeDtypeStruct((2, NA, width), jnp.float32),
            jax.ShapeDtypeStruct((2, NA, DEGW), jnp.float32),
        ],
        mesh=mesh,
        compiler_params=pltpu.CompilerParams(use_tc_tiling_on_sc=False),
        scratch_types=[
            pltpu.VMEM((KCH, CHUNK), jnp.int32),
            pltpu.VMEM((KCH, CHUNK), jnp.int32),
            pltpu.VMEM((CHUNK, width), jnp.float32),
            pltpu.VMEM((CHUNK, width), jnp.float32),
            pltpu.VMEM((CHUNK, DEGW), jnp.float32),
            pltpu.VMEM_SHARED((NA, width), jnp.float32),
            pltpu.VMEM_SHARED((NA, DEGW), jnp.float32),
            pltpu.SemaphoreType.DMA,
            pltpu.SemaphoreType.DMA,
        ],
    )
    def sc_kernel(rows_hbm, srcm, dstm, zeros_hbm, zerosd_hbm, ones_hbm,
                  out_hbm, outd_hbm,
                  src_v, dst_v, buf0, buf1, ones_v, acc, accd, sem0, sem1):
        cid = lax.axis_index("c")
        sid = lax.axis_index("s")
        w = cid * 16 + sid
        pltpu.sync_copy(zeros_hbm.at[pl.ds(sid * ROWS_PER_TILE, ROWS_PER_TILE)],
                        acc.at[pl.ds(sid * ROWS_PER_TILE, ROWS_PER_TILE)])
        pltpu.sync_copy(zerosd_hbm.at[pl.ds(sid * ROWS_PER_TILE, ROWS_PER_TILE)],
                        accd.at[pl.ds(sid * ROWS_PER_TILE, ROWS_PER_TILE)])
        pltpu.sync_copy(srcm.at[:, w], src_v)
        pltpu.sync_copy(dstm.at[:, w], dst_v)
        pltpu.sync_copy(ones_hbm, ones_v)
        plsc.subcore_barrier()

        pltpu.async_copy(rows_hbm.at[src_v.at[0]], buf0, sem0)

        def body2(k, carry):
            j0 = 2 * k
            j1 = j0 + 1
            j2 = jnp.minimum(j0 + 2, KCH - 1)
            pltpu.make_async_copy(rows_hbm.at[src_v.at[j0]], buf0, sem0).wait()
            pltpu.async_copy(rows_hbm.at[src_v.at[j1]], buf1, sem1)
            pltpu.sync_copy(buf0, acc.at[dst_v.at[j0]], add=True)
            pltpu.sync_copy(ones_v, accd.at[dst_v.at[j0]], add=True)
            pltpu.make_async_copy(rows_hbm.at[src_v.at[j1]], buf1, sem1).wait()
            pltpu.async_copy(rows_hbm.at[src_v.at[j2]], buf0, sem0)
            pltpu.sync_copy(buf1, acc.at[dst_v.at[j1]], add=True)
            pltpu.sync_copy(ones_v, accd.at[dst_v.at[j1]], add=True)
            return carry

        lax.fori_loop(0, KCH // 2, body2, 0)
        pltpu.make_async_copy(rows_hbm.at[src_v.at[0]], buf0, sem0).wait()
        plsc.subcore_barrier()
        pltpu.sync_copy(acc.at[pl.ds(sid * ROWS_PER_TILE, ROWS_PER_TILE)],
                        out_hbm.at[cid, pl.ds(sid * ROWS_PER_TILE, ROWS_PER_TILE)])
        pltpu.sync_copy(accd.at[pl.ds(sid * ROWS_PER_TILE, ROWS_PER_TILE)],
                        outd_hbm.at[cid, pl.ds(sid * ROWS_PER_TILE, ROWS_PER_TILE)])

    return sc_kernel


def _make_sc_segsum(width):
    mesh = plsc.VectorSubcoreMesh(core_axis_name="c", subcore_axis_name="s")

    @functools.partial(
        pl.kernel,
        out_type=jax.ShapeDtypeStruct((2, NA, width), jnp.float32),
        mesh=mesh,
        compiler_params=pltpu.CompilerParams(use_tc_tiling_on_sc=False),
        scratch_types=[
            pltpu.VMEM((KCH, CHUNK), jnp.int32),
            pltpu.VMEM((KCH, CHUNK), jnp.int32),
            pltpu.VMEM((CHUNK, width), jnp.float32),
            pltpu.VMEM((CHUNK, width), jnp.float32),
            pltpu.VMEM_SHARED((NA, width), jnp.float32),
            pltpu.SemaphoreType.DMA,
            pltpu.SemaphoreType.DMA,
        ],
    )
    def sc_kernel(rows_hbm, srcm, dstm, zeros_hbm, out_hbm,
                  src_v, dst_v, buf0, buf1, acc, sem0, sem1):
        cid = lax.axis_index("c")
        sid = lax.axis_index("s")
        w = cid * 16 + sid
        pltpu.sync_copy(zeros_hbm.at[pl.ds(sid * ROWS_PER_TILE, ROWS_PER_TILE)],
                        acc.at[pl.ds(sid * ROWS_PER_TILE, ROWS_PER_TILE)])
        pltpu.sync_copy(srcm.at[:, w], src_v)
        pltpu.sync_copy(dstm.at[:, w], dst_v)
        plsc.subcore_barrier()

        pltpu.async_copy(rows_hbm.at[src_v.at[0]], buf0, sem0)

        def body2(k, carry):
            j0 = 2 * k
            j1 = j0 + 1
            j2 = jnp.minimum(j0 + 2, KCH - 1)
            pltpu.make_async_copy(rows_hbm.at[src_v.at[j0]], buf0, sem0).wait()
            pltpu.async_copy(rows_hbm.at[src_v.at[j1]], buf1, sem1)
            pltpu.sync_copy(buf0, acc.at[dst_v.at[j0]], add=True)
            pltpu.make_async_copy(rows_hbm.at[src_v.at[j1]], buf1, sem1).wait()
            pltpu.async_copy(rows_hbm.at[src_v.at[j2]], buf0, sem0)
            pltpu.sync_copy(buf1, acc.at[dst_v.at[j1]], add=True)
            return carry

        lax.fori_loop(0, KCH // 2, body2, 0)
        pltpu.make_async_copy(rows_hbm.at[src_v.at[0]], buf0, sem0).wait()
        plsc.subcore_barrier()
        pltpu.sync_copy(acc.at[pl.ds(sid * ROWS_PER_TILE, ROWS_PER_TILE)],
                        out_hbm.at[cid, pl.ds(sid * ROWS_PER_TILE, ROWS_PER_TILE)])

    return sc_kernel


def _tc2_body(h0_ref, p1_ref, pd_ref, w1, b1, w2, b2, h1_ref, st_ref, cnt_ref):
    st = p1_ref[0] + p1_ref[1]
    st_ref[...] = st
    cnt_ref[...] = pd_ref[0] + pd_ref[1]
    pre = h0_ref[...] + st[:, :HID]
    t = _relu(_dot(pre, w1[...]) + b1[...])
    h1_ref[...] = _relu(_dot(t, w2[...]) + b2[...])


def _tc2(h0, part1, partd, w1, b1, w2, b2):
    grid = (N // BLK,)
    wspec = pl.BlockSpec((HID, HID), lambda i: (0, 0))
    bspec = pl.BlockSpec((1, HID), lambda i: (0, 0))
    return pl.pallas_call(
        _tc2_body,
        grid=grid,
        in_specs=[
            pl.BlockSpec((BLK, HID), lambda i: (i, 0)),
            pl.BlockSpec((2, BLK, ROW1), lambda i: (0, i, 0)),
            pl.BlockSpec((2, BLK, DEGW), lambda i: (0, i, 0)),
            wspec, bspec, wspec, bspec,
        ],
        out_specs=[
            pl.BlockSpec((BLK, HID), lambda i: (i, 0)),
            pl.BlockSpec((BLK, ROW1), lambda i: (i, 0)),
            pl.BlockSpec((BLK, DEGW), lambda i: (i, 0)),
        ],
        out_shape=[
            jax.ShapeDtypeStruct((N, HID), jnp.float32),
            jax.ShapeDtypeStruct((N, ROW1), jnp.float32),
            jax.ShapeDtypeStruct((N, DEGW), jnp.float32),
        ],
    )(h0, part1, partd, w1, b1, w2, b2)


def _tc3_body(h0_ref, h1_ref, st_ref, cnt_ref, p2_ref, z_ref, *refs):
    (g1w1, g1b1, g1w2, g1b2,
     dw1, db1, dw2, db2, dw3, db3, dw4, db4,
     slw, slb, srw, srb, stw, stb,
     fw1, fb1, fw2, fb2, fw3, fb3,
     mw1, mb1, mw2, mb2, mw3, mb3,
     gw1, gb1, gw2, gb2, gw3, gb3,
     nw1, nb1, nw2, nb2, nw3, nb3, nw4, nb4,
     l1_ref, scal_ref, h0p_ref) = refs

    h0 = h0_ref[...]
    h1 = h1_ref[...]
    st = st_ref[...]
    s3 = p2_ref[0] + p2_ref[1]

    pre = h1 + s3
    t = _relu(_dot(pre, g1w1[...]) + g1b1[...])
    l1 = _dot(t, g1w2[...]) + g1b2[...]
    l1_ref[...] = l1

    t = _relu(_dot(l1, dw1[...]) + db1[...])
    t = _relu(_dot(t, dw2[...]) + db2[...])
    t = _relu(_dot(t, dw3[...]) + db3[...])
    deg = _relu(_dot(t, dw4[...]) + db4[...])

    s1 = st[:, :HID]
    s2 = st[:, HID:2 * HID]
    cnt = cnt_ref[...][:, 0:1]
    denom = jnp.maximum(cnt, 1.0)
    mean_n = s1 / denom
    mean_sq = s2 / denom
    std_raw = jnp.sqrt(_relu(mean_sq - mean_n * mean_n) + 1e-5)
    mean_neigh = (_dot(mean_n, slw[...]) + slb[...]
                  + _dot(h0, srw[...]) + srb[...])
    s = _dot(std_raw, stw[...]) + stb[...]

    t = _relu(_dot(l1, fw1[...]) + fb1[...])
    t = _relu(_dot(t, fw2[...]) + fb2[...])
    h0p_ref[...] = _dot(t, fw3[...]) + fb3[...]

    t = _relu(_dot(l1, mw1[...]) + mb1[...])
    t = _relu(_dot(t, mw2[...]) + mb2[...])
    g_mean = _dot(t, mw3[...]) + mb3[...]
    t = _relu(_dot(l1, gw1[...]) + gb1[...])
    t = _relu(_dot(t, gw2[...]) + gb2[...])
    g_sigma = _dot(t, gw3[...]) + gb3[...]
    escale = jnp.exp(g_sigma)

    def gen(var):
        t = _relu(_dot(var, nw1[...]) + nb1[...])
        t = _relu(_dot(t, nw2[...]) + nb2[...])
        t = _relu(_dot(t, nw3[...]) + nb3[...])
        return _dot(t, nw4[...]) + nb4[...]

    n0 = gen(g_mean + escale * z_ref[0])
    n1 = gen(g_mean + escale * z_ref[1])
    gen_mean = 0.5 * (n0 + n1)
    u = 0.5 * jnp.abs(n0 - n1)

    ss = jnp.sum(s * s, axis=1, keepdims=True)
    uu = jnp.sum(u * u, axis=1, keepdims=True)
    us = jnp.sum(u * s, axis=1, keepdims=True)
    det_t = 1.0 + ss
    det_g = 1.0 + uu
    trace = HID + ss - (uu + us * us) / det_g
    d = gen_mean - mean_neigh
    zq = (jnp.sum(d * d, axis=1, keepdims=True)
          - jnp.sum(u * d, axis=1, keepdims=True) ** 2 / det_g)
    kl = 0.5 * (jnp.log(det_g / det_t) - HID + trace + zq)
    br = deg.shape[0]
    scal_ref[...] = jnp.concatenate(
        [deg, det_t, det_g, trace, zq, kl, jnp.zeros((br, 2), jnp.float32)],
        axis=1)


def _tc3(h0, h1, st, cnt, part2, z, wlist):
    grid = (N // BLK,)

    def fullspec(a):
        nd = a.ndim
        return pl.BlockSpec(a.shape, lambda i, _nd=nd: (0,) * _nd)

    in_specs = [
        pl.BlockSpec((BLK, HID), lambda i: (i, 0)),
        pl.BlockSpec((BLK, HID), lambda i: (i, 0)),
        pl.BlockSpec((BLK, ROW1), lambda i: (i, 0)),
        pl.BlockSpec((BLK, DEGW), lambda i: (i, 0)),
        pl.BlockSpec((2, BLK, HID), lambda i: (0, i, 0)),
        pl.BlockSpec((2, BLK, HID), lambda i: (0, i, 0)),
    ] + [fullspec(a) for a in wlist]
    return pl.pallas_call(
        _tc3_body,
        grid=grid,
        in_specs=in_specs,
        out_specs=[
            pl.BlockSpec((BLK, HID), lambda i: (i, 0)),
            pl.BlockSpec((BLK, 8), lambda i: (i, 0)),
            pl.BlockSpec((BLK, IN_DIM), lambda i: (i, 0)),
        ],
        out_shape=[
            jax.ShapeDtypeStruct((N, HID), jnp.float32),
            jax.ShapeDtypeStruct((N, 8), jnp.float32),
            jax.ShapeDtypeStruct((N, IN_DIM), jnp.float32),
        ],
    )(h0, h1, st, cnt, part2, z, *wlist)


def _b2(b):
    return b.reshape(1, -1)


def kernel(x, edge_index, params):
    src = edge_index[0].astype(jnp.int32)
    dst = edge_index[1].astype(jnp.int32)
    srcm = jnp.concatenate([src, _PAD_SRC_TAIL]).reshape(KCH, NW, CHUNK)
    dstm = jnp.concatenate([dst, _PAD_DST_TAIL]).reshape(KCH, NW, CHUNK)

    p = params
    lw, lb = p["lin"]
    h0, h0e = _tc1(x, lw, _b2(lb))

    part1, partd = _make_sc_segsum_deg(ROW1)(
        h0e, srcm, dstm, _ZEROS_ROW1, _ZEROS_DEG, _ONES_DEG)

    (g0w1, g0b1), (g0w2, g0b2) = p["gin"][0]
    h1, st, cnt = _tc2(h0, part1, partd, g0w1, _b2(g0b1), g0w2, _b2(g0b2))

    part2 = _make_sc_segsum(HID)(h1, srcm, dstm, _ZEROS_HID)

    wlist = []
    for (w, b) in p["gin"][1]:
        wlist += [w, _b2(b)]
    for (w, b) in p["deg"]:
        wlist += [w, _b2(b)]
    for key in ("sage_l", "sage_r", "std_lin"):
        w, b = p[key]
        wlist += [w, _b2(b)]
    for key in ("feat", "mlp_mean", "mlp_sigma", "gen"):
        for (w, b) in p[key]:
            wlist += [w, _b2(b)]

    l1, scal, h0p = _tc3(h0, h1, st, cnt, part2, _Z, wlist)

    return (h0, l1, scal[:, 0:1], (h0p,),
            ((scal[:, 1], scal[:, 2], scal[:, 3], scal[:, 4], scal[:, 5]),))


# speedup vs baseline: 2.0322x; 1.0498x over previous
import functools

import jax
import jax.numpy as jnp
from jax import lax
from jax.experimental import pallas as pl
from jax.experimental.pallas import tpu as pltpu
from jax.experimental.pallas import tpu_sc as plsc

N = 10000
E = 320000
IN_DIM = 128
HID = 32
NA = 10240
CHUNK = 128
NW = 32
KCH = 80
SLOT = KCH * CHUNK
EP = NW * SLOT
REAL_PW = E // NW
PAD_PW = SLOT - REAL_PW
ROW1 = 2 * HID
DEGW = 8
ROWS_PER_TILE = NA // 16
BLK = 400

_p = jnp.arange(EP - E, dtype=jnp.int32)
_PAD_SRC_TAIL = _p % N
_PAD_DST_TAIL = N + _p % (NA - N)

_Z = jax.random.normal(jax.random.fold_in(jax.random.key(1), 0),
                       (2, N, HID), jnp.float32)
_ZEROS_ROW1 = jnp.zeros((NA, ROW1), jnp.float32)
_ZEROS_HID = jnp.zeros((NA, HID), jnp.float32)
_ZEROS_DEG = jnp.zeros((NA, DEGW), jnp.float32)
_ONES_DEG = jnp.ones((CHUNK, DEGW), jnp.float32)


def _relu(v):
    return jnp.maximum(v, 0.0)


def _dot(a, w):
    return jnp.dot(a, w, preferred_element_type=jnp.float32)


def _tc1_body(x_ref, w_ref, b_ref, h0_ref, h0e_ref):
    h0 = _dot(x_ref[...], w_ref[...]) + b_ref[...]
    h0_ref[...] = h0
    h0e_ref[...] = jnp.concatenate([h0, h0 * h0], axis=1)


def _tc1(x, w, b):
    grid = (N // BLK,)
    return pl.pallas_call(
        _tc1_body,
        grid=grid,
        in_specs=[
            pl.BlockSpec((BLK, IN_DIM), lambda i: (i, 0)),
            pl.BlockSpec((IN_DIM, HID), lambda i: (0, 0)),
            pl.BlockSpec((1, HID), lambda i: (0, 0)),
        ],
        out_specs=[
            pl.BlockSpec((BLK, HID), lambda i: (i, 0)),
            pl.BlockSpec((BLK, ROW1), lambda i: (i, 0)),
        ],
        out_shape=[
            jax.ShapeDtypeStruct((N, HID), jnp.float32),
            jax.ShapeDtypeStruct((N, ROW1), jnp.float32),
        ],
    )(x, w, b)


def _make_sc_segsum_deg(width):
    mesh = plsc.VectorSubcoreMesh(core_axis_name="c", subcore_axis_name="s")

    @functools.partial(
        pl.kernel,
        out_type=[
            jax.ShapeDtypeStruct((2, NA, width), jnp.float32),
            jax.ShapeDtypeStruct((2, NA, DEGW), jnp.float32),
        ],
        mesh=mesh,
        compiler_params=pltpu.CompilerParams(use_tc_tiling_on_sc=False),
        scratch_types=[
            pltpu.VMEM((KCH, CHUNK), jnp.int32),
            pltpu.VMEM((KCH, CHUNK), jnp.int32),
            pltpu.VMEM((CHUNK, width), jnp.float32),
            pltpu.VMEM((CHUNK, width), jnp.float32),
            pltpu.VMEM((CHUNK, DEGW), jnp.float32),
            pltpu.VMEM_SHARED((NA, width), jnp.float32),
            pltpu.VMEM_SHARED((NA, DEGW), jnp.float32),
            pltpu.SemaphoreType.DMA,
            pltpu.SemaphoreType.DMA,
        ],
    )
    def sc_kernel(rows_hbm, srcm, dstm, zeros_hbm, zerosd_hbm, ones_hbm,
                  out_hbm, outd_hbm,
                  src_v, dst_v, buf0, buf1, ones_v, acc, accd, sem0, sem1):
        cid = lax.axis_index("c")
        sid = lax.axis_index("s")
        w = cid * 16 + sid
        pltpu.sync_copy(zeros_hbm.at[pl.ds(sid * ROWS_PER_TILE, ROWS_PER_TILE)],
                        acc.at[pl.ds(sid * ROWS_PER_TILE, ROWS_PER_TILE)])
        pltpu.sync_copy(zerosd_hbm.at[pl.ds(sid * ROWS_PER_TILE, ROWS_PER_TILE)],
                        accd.at[pl.ds(sid * ROWS_PER_TILE, ROWS_PER_TILE)])
        pltpu.sync_copy(srcm.at[:, w], src_v)
        pltpu.sync_copy(dstm.at[:, w], dst_v)
        pltpu.sync_copy(ones_hbm, ones_v)
        plsc.subcore_barrier()

        pltpu.async_copy(rows_hbm.at[src_v.at[0]], buf0, sem0)

        def body2(k, carry):
            j0 = 2 * k
            j1 = j0 + 1
            j2 = jnp.minimum(j0 + 2, KCH - 1)
            pltpu.make_async_copy(rows_hbm.at[src_v.at[j0]], buf0, sem0).wait()
            pltpu.async_copy(rows_hbm.at[src_v.at[j1]], buf1, sem1)
            pltpu.sync_copy(buf0, acc.at[dst_v.at[j0]], add=True)
            pltpu.sync_copy(ones_v, accd.at[dst_v.at[j0]], add=True)
            pltpu.make_async_copy(rows_hbm.at[src_v.at[j1]], buf1, sem1).wait()
            pltpu.async_copy(rows_hbm.at[src_v.at[j2]], buf0, sem0)
            pltpu.sync_copy(buf1, acc.at[dst_v.at[j1]], add=True)
            pltpu.sync_copy(ones_v, accd.at[dst_v.at[j1]], add=True)
            return carry

        lax.fori_loop(0, KCH // 2, body2, 0)
        pltpu.make_async_copy(rows_hbm.at[src_v.at[0]], buf0, sem0).wait()
        plsc.subcore_barrier()
        pltpu.sync_copy(acc.at[pl.ds(sid * ROWS_PER_TILE, ROWS_PER_TILE)],
                        out_hbm.at[cid, pl.ds(sid * ROWS_PER_TILE, ROWS_PER_TILE)])
        pltpu.sync_copy(accd.at[pl.ds(sid * ROWS_PER_TILE, ROWS_PER_TILE)],
                        outd_hbm.at[cid, pl.ds(sid * ROWS_PER_TILE, ROWS_PER_TILE)])

    return sc_kernel


def _make_sc_segsum(width):
    mesh = plsc.VectorSubcoreMesh(core_axis_name="c", subcore_axis_name="s")

    @functools.partial(
        pl.kernel,
        out_type=jax.ShapeDtypeStruct((2, NA, width), jnp.float32),
        mesh=mesh,
        compiler_params=pltpu.CompilerParams(use_tc_tiling_on_sc=False),
        scratch_types=[
            pltpu.VMEM((KCH, CHUNK), jnp.int32),
            pltpu.VMEM((KCH, CHUNK), jnp.int32),
            pltpu.VMEM((CHUNK, width), jnp.float32),
            pltpu.VMEM((CHUNK, width), jnp.float32),
            pltpu.VMEM_SHARED((NA, width), jnp.float32),
            pltpu.SemaphoreType.DMA,
            pltpu.SemaphoreType.DMA,
        ],
    )
    def sc_kernel(rows_hbm, srcm, dstm, zeros_hbm, out_hbm,
                  src_v, dst_v, buf0, buf1, acc, sem0, sem1):
        cid = lax.axis_index("c")
        sid = lax.axis_index("s")
        w = cid * 16 + sid
        pltpu.sync_copy(zeros_hbm.at[pl.ds(sid * ROWS_PER_TILE, ROWS_PER_TILE)],
                        acc.at[pl.ds(sid * ROWS_PER_TILE, ROWS_PER_TILE)])
        pltpu.sync_copy(srcm.at[:, w], src_v)
        pltpu.sync_copy(dstm.at[:, w], dst_v)
        plsc.subcore_barrier()

        pltpu.async_copy(rows_hbm.at[src_v.at[0]], buf0, sem0)

        def body2(k, carry):
            j0 = 2 * k
            j1 = j0 + 1
            j2 = jnp.minimum(j0 + 2, KCH - 1)
            pltpu.make_async_copy(rows_hbm.at[src_v.at[j0]], buf0, sem0).wait()
            pltpu.async_copy(rows_hbm.at[src_v.at[j1]], buf1, sem1)
            pltpu.sync_copy(buf0, acc.at[dst_v.at[j0]], add=True)
            pltpu.make_async_copy(rows_hbm.at[src_v.at[j1]], buf1, sem1).wait()
            pltpu.async_copy(rows_hbm.at[src_v.at[j2]], buf0, sem0)
            pltpu.sync_copy(buf1, acc.at[dst_v.at[j1]], add=True)
            return carry

        lax.fori_loop(0, KCH // 2, body2, 0)
        pltpu.make_async_copy(rows_hbm.at[src_v.at[0]], buf0, sem0).wait()
        plsc.subcore_barrier()
        pltpu.sync_copy(acc.at[pl.ds(sid * ROWS_PER_TILE, ROWS_PER_TILE)],
                        out_hbm.at[cid, pl.ds(sid * ROWS_PER_TILE, ROWS_PER_TILE)])

    return sc_kernel


def _tc2_body(h0_ref, p1_ref, pd_ref, w1, b1, w2, b2, h1_ref, st_ref, cnt_ref):
    st = p1_ref[0] + p1_ref[1]
    st_ref[...] = st
    cnt_ref[...] = pd_ref[0] + pd_ref[1]
    pre = h0_ref[...] + st[:, :HID]
    t = _relu(_dot(pre, w1[...]) + b1[...])
    h1_ref[...] = _relu(_dot(t, w2[...]) + b2[...])


def _tc2(h0, part1, partd, w1, b1, w2, b2):
    grid = (N // BLK,)
    wspec = pl.BlockSpec((HID, HID), lambda i: (0, 0))
    bspec = pl.BlockSpec((1, HID), lambda i: (0, 0))
    return pl.pallas_call(
        _tc2_body,
        grid=grid,
        in_specs=[
            pl.BlockSpec((BLK, HID), lambda i: (i, 0)),
            pl.BlockSpec((2, BLK, ROW1), lambda i: (0, i, 0)),
            pl.BlockSpec((2, BLK, DEGW), lambda i: (0, i, 0)),
            wspec, bspec, wspec, bspec,
        ],
        out_specs=[
            pl.BlockSpec((BLK, HID), lambda i: (i, 0)),
            pl.BlockSpec((BLK, ROW1), lambda i: (i, 0)),
            pl.BlockSpec((BLK, DEGW), lambda i: (i, 0)),
        ],
        out_shape=[
            jax.ShapeDtypeStruct((N, HID), jnp.float32),
            jax.ShapeDtypeStruct((N, ROW1), jnp.float32),
            jax.ShapeDtypeStruct((N, DEGW), jnp.float32),
        ],
    )(h0, part1, partd, w1, b1, w2, b2)


def _tc3_body(h0_ref, h1_ref, st_ref, cnt_ref, p2_ref, z_ref, *refs):
    (g1w1, g1b1, g1w2, g1b2,
     hw1, hb1, hw2, hb2, hw3, hb3, dw4, db4,
     sw, sb,
     nw1, nb1, nw2, nb2, nw3, nb3, nw4, nb4,
     l1_ref, scal_ref, h0p_ref) = refs

    h0 = h0_ref[...]
    h1 = h1_ref[...]
    st = st_ref[...]
    s3 = p2_ref[0] + p2_ref[1]

    pre = h1 + s3
    t = _relu(_dot(pre, g1w1[...]) + g1b1[...])
    l1 = _dot(t, g1w2[...]) + g1b2[...]
    l1_ref[...] = l1

    t = _relu(_dot(l1, hw1[...]) + hb1[...])
    t = _relu(_dot(t, hw2[...]) + hb2[...])
    t3 = _dot(t, hw3[...]) + hb3[...]
    deg = _relu(_dot(_relu(t3[:, :HID]), dw4[...]) + db4[...])
    h0p_ref[...] = t3[:, HID:HID + IN_DIM]
    g_mean = t3[:, HID + IN_DIM:2 * HID + IN_DIM]
    g_sigma = t3[:, 2 * HID + IN_DIM:3 * HID + IN_DIM]
    escale = jnp.exp(g_sigma)

    s1 = st[:, :HID]
    s2 = st[:, HID:2 * HID]
    cnt = cnt_ref[...][:, 0:1]
    denom = jnp.maximum(cnt, 1.0)
    mean_n = s1 / denom
    mean_sq = s2 / denom
    std_raw = jnp.sqrt(_relu(mean_sq - mean_n * mean_n) + 1e-5)
    sg = _dot(jnp.concatenate([mean_n, h0, std_raw], axis=1), sw[...]) + sb[...]
    mean_neigh = sg[:, :HID]
    s = sg[:, HID:2 * HID]

    gi = jnp.concatenate([g_mean + escale * z_ref[0],
                          g_mean + escale * z_ref[1]], axis=0)
    t = _relu(_dot(gi, nw1[...]) + nb1[...])
    t = _relu(_dot(t, nw2[...]) + nb2[...])
    t = _relu(_dot(t, nw3[...]) + nb3[...])
    go = _dot(t, nw4[...]) + nb4[...]
    br = h0.shape[0]
    n0 = go[:br]
    n1 = go[br:]
    gen_mean = 0.5 * (n0 + n1)
    u = 0.5 * jnp.abs(n0 - n1)

    ss = jnp.sum(s * s, axis=1, keepdims=True)
    uu = jnp.sum(u * u, axis=1, keepdims=True)
    us = jnp.sum(u * s, axis=1, keepdims=True)
    det_t = 1.0 + ss
    det_g = 1.0 + uu
    trace = HID + ss - (uu + us * us) / det_g
    d = gen_mean - mean_neigh
    zq = (jnp.sum(d * d, axis=1, keepdims=True)
          - jnp.sum(u * d, axis=1, keepdims=True) ** 2 / det_g)
    kl = 0.5 * (jnp.log(det_g / det_t) - HID + trace + zq)
    br = deg.shape[0]
    scal_ref[...] = jnp.concatenate(
        [deg, det_t, det_g, trace, zq, kl, jnp.zeros((br, 2), jnp.float32)],
        axis=1)


def _tc3(h0, h1, st, cnt, part2, z, wlist):
    grid = (N // BLK,)

    def fullspec(a):
        nd = a.ndim
        return pl.BlockSpec(a.shape, lambda i, _nd=nd: (0,) * _nd)

    in_specs = [
        pl.BlockSpec((BLK, HID), lambda i: (i, 0)),
        pl.BlockSpec((BLK, HID), lambda i: (i, 0)),
        pl.BlockSpec((BLK, ROW1), lambda i: (i, 0)),
        pl.BlockSpec((BLK, DEGW), lambda i: (i, 0)),
        pl.BlockSpec((2, BLK, HID), lambda i: (0, i, 0)),
        pl.BlockSpec((2, BLK, HID), lambda i: (0, i, 0)),
    ] + [fullspec(a) for a in wlist]
    return pl.pallas_call(
        _tc3_body,
        grid=grid,
        in_specs=in_specs,
        out_specs=[
            pl.BlockSpec((BLK, HID), lambda i: (i, 0)),
            pl.BlockSpec((BLK, 8), lambda i: (i, 0)),
            pl.BlockSpec((BLK, IN_DIM), lambda i: (i, 0)),
        ],
        out_shape=[
            jax.ShapeDtypeStruct((N, HID), jnp.float32),
            jax.ShapeDtypeStruct((N, 8), jnp.float32),
            jax.ShapeDtypeStruct((N, IN_DIM), jnp.float32),
        ],
    )(h0, h1, st, cnt, part2, z, *wlist)


def _b2(b):
    return b.reshape(1, -1)


def _bdiag(blocks):
    rows = sum(b.shape[0] for b in blocks)
    cols = sum(b.shape[1] for b in blocks)
    out = jnp.zeros((rows, cols), jnp.float32)
    r = c = 0
    for b in blocks:
        out = out.at[r:r + b.shape[0], c:c + b.shape[1]].set(b)
        r += b.shape[0]
        c += b.shape[1]
    return out


def kernel(x, edge_index, params):
    src = edge_index[0].astype(jnp.int32)
    dst = edge_index[1].astype(jnp.int32)
    srcm = jnp.concatenate([src, _PAD_SRC_TAIL]).reshape(KCH, NW, CHUNK)
    dstm = jnp.concatenate([dst, _PAD_DST_TAIL]).reshape(KCH, NW, CHUNK)

    p = params
    lw, lb = p["lin"]
    h0, h0e = _tc1(x, lw, _b2(lb))

    part1, partd = _make_sc_segsum_deg(ROW1)(
        h0e, srcm, dstm, _ZEROS_ROW1, _ZEROS_DEG, _ONES_DEG)

    (g0w1, g0b1), (g0w2, g0b2) = p["gin"][0]
    h1, st, cnt = _tc2(h0, part1, partd, g0w1, _b2(g0b1), g0w2, _b2(g0b2))

    part2 = _make_sc_segsum(HID)(h1, srcm, dstm, _ZEROS_HID)

    dg, ft, mm, ms = p["deg"], p["feat"], p["mlp_mean"], p["mlp_sigma"]
    hw1 = jnp.concatenate([dg[0][0], ft[0][0], mm[0][0], ms[0][0]], axis=1)
    hb1 = jnp.concatenate([dg[0][1], ft[0][1], mm[0][1], ms[0][1]]).reshape(1, -1)
    hw2 = _bdiag([dg[1][0], ft[1][0], mm[1][0], ms[1][0]])
    hb2 = jnp.concatenate([dg[1][1], ft[1][1], mm[1][1], ms[1][1]]).reshape(1, -1)
    hw3 = _bdiag([dg[2][0], ft[2][0], mm[2][0], ms[2][0]])
    hb3 = jnp.concatenate([dg[2][1], ft[2][1], mm[2][1], ms[2][1]]).reshape(1, -1)
    slw, slb = p["sage_l"]
    srw, srb = p["sage_r"]
    stw, stb = p["std_lin"]
    z32 = jnp.zeros((HID, HID), jnp.float32)
    sw = jnp.concatenate([
        jnp.concatenate([slw, z32], axis=1),
        jnp.concatenate([srw, z32], axis=1),
        jnp.concatenate([z32, stw], axis=1)], axis=0)
    sb = jnp.concatenate([slb + srb, stb]).reshape(1, -1)

    (g1w1, g1b1), (g1w2, g1b2) = p["gin"][1]
    wlist = [g1w1, _b2(g1b1), g1w2, _b2(g1b2),
             hw1, hb1, hw2, hb2, hw3, hb3, dg[3][0], _b2(dg[3][1]), sw, sb]
    for (w, b) in p["gen"]:
        wlist += [w, _b2(b)]

    l1, scal, h0p = _tc3(h0, h1, st, cnt, part2, _Z, wlist)

    return (h0, l1, scal[:, 0:1], (h0p,),
            ((scal[:, 1], scal[:, 2], scal[:, 3], scal[:, 4], scal[:, 5]),))


# speedup vs baseline: 2.6343x; 1.2963x over previous
import functools

import jax
import jax.numpy as jnp
from jax import lax
from jax.experimental import pallas as pl
from jax.experimental.pallas import tpu as pltpu
from jax.experimental.pallas import tpu_sc as plsc

N = 10000
E = 320000
IN_DIM = 128
HID = 32
NA = 10240
CHUNK = 256
NW = 32
KCH = 40
SLOT = KCH * CHUNK
EP = NW * SLOT
REAL_PW = E // NW
PAD_PW = SLOT - REAL_PW
ROW1 = 2 * HID
DEGW = 8
ROWS_PER_TILE = NA // 16
BLK = 2000

_p = jnp.arange(EP - E, dtype=jnp.int32)
_PAD_SRC_TAIL = _p % N
_PAD_DST_TAIL = N + _p % (NA - N)

_Z = jax.random.normal(jax.random.fold_in(jax.random.key(1), 0),
                       (2, N, HID), jnp.float32)
_ZEROS_ROW1 = jnp.zeros((NA, ROW1), jnp.float32)
_ZEROS_HID = jnp.zeros((NA, HID), jnp.float32)
_ZEROS_DEG = jnp.zeros((NA, DEGW), jnp.float32)
_ONES_DEG = jnp.ones((CHUNK, DEGW), jnp.float32)


def _relu(v):
    return jnp.maximum(v, 0.0)


def _dot(a, w):
    return jnp.dot(a, w, preferred_element_type=jnp.float32)


def _tc1_body(x_ref, w_ref, b_ref, h0_ref, h0e_ref):
    h0 = _dot(x_ref[...], w_ref[...]) + b_ref[...]
    h0_ref[...] = h0
    h0e_ref[...] = jnp.concatenate([h0, h0 * h0], axis=1)


def _tc1(x, w, b):
    grid = (N // BLK,)
    return pl.pallas_call(
        _tc1_body,
        grid=grid,
        in_specs=[
            pl.BlockSpec((BLK, IN_DIM), lambda i: (i, 0)),
            pl.BlockSpec((IN_DIM, HID), lambda i: (0, 0)),
            pl.BlockSpec((1, HID), lambda i: (0, 0)),
        ],
        out_specs=[
            pl.BlockSpec((BLK, HID), lambda i: (i, 0)),
            pl.BlockSpec((BLK, ROW1), lambda i: (i, 0)),
        ],
        out_shape=[
            jax.ShapeDtypeStruct((N, HID), jnp.float32),
            jax.ShapeDtypeStruct((N, ROW1), jnp.float32),
        ],
    )(x, w, b)


def _make_sc_segsum_deg(width):
    mesh = plsc.VectorSubcoreMesh(core_axis_name="c", subcore_axis_name="s")

    @functools.partial(
        pl.kernel,
        out_type=[
            jax.ShapeDtypeStruct((2, NA, width), jnp.float32),
            jax.ShapeDtypeStruct((2, NA, DEGW), jnp.float32),
        ],
        mesh=mesh,
        compiler_params=pltpu.CompilerParams(use_tc_tiling_on_sc=False),
        scratch_types=[
            pltpu.VMEM((KCH, CHUNK), jnp.int32),
            pltpu.VMEM((KCH, CHUNK), jnp.int32),
            pltpu.VMEM((CHUNK, width), jnp.float32),
            pltpu.VMEM((CHUNK, width), jnp.float32),
            pltpu.VMEM((CHUNK, DEGW), jnp.float32),
            pltpu.VMEM_SHARED((NA, width), jnp.float32),
            pltpu.VMEM_SHARED((NA, DEGW), jnp.float32),
            pltpu.SemaphoreType.DMA,
            pltpu.SemaphoreType.DMA,
        ],
    )
    def sc_kernel(rows_hbm, srcm, dstm, zeros_hbm, zerosd_hbm, ones_hbm,
                  out_hbm, outd_hbm,
                  src_v, dst_v, buf0, buf1, ones_v, acc, accd, sem0, sem1):
        cid = lax.axis_index("c")
        sid = lax.axis_index("s")
        w = cid * 16 + sid
        pltpu.sync_copy(zeros_hbm.at[pl.ds(sid * ROWS_PER_TILE, ROWS_PER_TILE)],
                        acc.at[pl.ds(sid * ROWS_PER_TILE, ROWS_PER_TILE)])
        pltpu.sync_copy(zerosd_hbm.at[pl.ds(sid * ROWS_PER_TILE, ROWS_PER_TILE)],
                        accd.at[pl.ds(sid * ROWS_PER_TILE, ROWS_PER_TILE)])
        pltpu.sync_copy(srcm.at[:, w], src_v)
        pltpu.sync_copy(dstm.at[:, w], dst_v)
        pltpu.sync_copy(ones_hbm, ones_v)
        plsc.subcore_barrier()

        pltpu.async_copy(rows_hbm.at[src_v.at[0]], buf0, sem0)

        def body2(k, carry):
            j0 = 2 * k
            j1 = j0 + 1
            j2 = jnp.minimum(j0 + 2, KCH - 1)
            pltpu.make_async_copy(rows_hbm.at[src_v.at[j0]], buf0, sem0).wait()
            pltpu.async_copy(rows_hbm.at[src_v.at[j1]], buf1, sem1)
            pltpu.sync_copy(buf0, acc.at[dst_v.at[j0]], add=True)
            pltpu.sync_copy(ones_v, accd.at[dst_v.at[j0]], add=True)
            pltpu.make_async_copy(rows_hbm.at[src_v.at[j1]], buf1, sem1).wait()
            pltpu.async_copy(rows_hbm.at[src_v.at[j2]], buf0, sem0)
            pltpu.sync_copy(buf1, acc.at[dst_v.at[j1]], add=True)
            pltpu.sync_copy(ones_v, accd.at[dst_v.at[j1]], add=True)
            return carry

        lax.fori_loop(0, KCH // 2, body2, 0)
        pltpu.make_async_copy(rows_hbm.at[src_v.at[0]], buf0, sem0).wait()
        plsc.subcore_barrier()
        pltpu.sync_copy(acc.at[pl.ds(sid * ROWS_PER_TILE, ROWS_PER_TILE)],
                        out_hbm.at[cid, pl.ds(sid * ROWS_PER_TILE, ROWS_PER_TILE)])
        pltpu.sync_copy(accd.at[pl.ds(sid * ROWS_PER_TILE, ROWS_PER_TILE)],
                        outd_hbm.at[cid, pl.ds(sid * ROWS_PER_TILE, ROWS_PER_TILE)])

    return sc_kernel


def _make_sc_segsum(width):
    mesh = plsc.VectorSubcoreMesh(core_axis_name="c", subcore_axis_name="s")

    @functools.partial(
        pl.kernel,
        out_type=jax.ShapeDtypeStruct((2, NA, width), jnp.float32),
        mesh=mesh,
        compiler_params=pltpu.CompilerParams(use_tc_tiling_on_sc=False),
        scratch_types=[
            pltpu.VMEM((KCH, CHUNK), jnp.int32),
            pltpu.VMEM((KCH, CHUNK), jnp.int32),
            pltpu.VMEM((CHUNK, width), jnp.float32),
            pltpu.VMEM((CHUNK, width), jnp.float32),
            pltpu.VMEM_SHARED((NA, width), jnp.float32),
            pltpu.SemaphoreType.DMA,
            pltpu.SemaphoreType.DMA,
        ],
    )
    def sc_kernel(rows_hbm, srcm, dstm, zeros_hbm, out_hbm,
                  src_v, dst_v, buf0, buf1, acc, sem0, sem1):
        cid = lax.axis_index("c")
        sid = lax.axis_index("s")
        w = cid * 16 + sid
        pltpu.sync_copy(zeros_hbm.at[pl.ds(sid * ROWS_PER_TILE, ROWS_PER_TILE)],
                        acc.at[pl.ds(sid * ROWS_PER_TILE, ROWS_PER_TILE)])
        pltpu.sync_copy(srcm.at[:, w], src_v)
        pltpu.sync_copy(dstm.at[:, w], dst_v)
        plsc.subcore_barrier()

        pltpu.async_copy(rows_hbm.at[src_v.at[0]], buf0, sem0)

        def body2(k, carry):
            j0 = 2 * k
            j1 = j0 + 1
            j2 = jnp.minimum(j0 + 2, KCH - 1)
            pltpu.make_async_copy(rows_hbm.at[src_v.at[j0]], buf0, sem0).wait()
            pltpu.async_copy(rows_hbm.at[src_v.at[j1]], buf1, sem1)
            pltpu.sync_copy(buf0, acc.at[dst_v.at[j0]], add=True)
            pltpu.make_async_copy(rows_hbm.at[src_v.at[j1]], buf1, sem1).wait()
            pltpu.async_copy(rows_hbm.at[src_v.at[j2]], buf0, sem0)
            pltpu.sync_copy(buf1, acc.at[dst_v.at[j1]], add=True)
            return carry

        lax.fori_loop(0, KCH // 2, body2, 0)
        pltpu.make_async_copy(rows_hbm.at[src_v.at[0]], buf0, sem0).wait()
        plsc.subcore_barrier()
        pltpu.sync_copy(acc.at[pl.ds(sid * ROWS_PER_TILE, ROWS_PER_TILE)],
                        out_hbm.at[cid, pl.ds(sid * ROWS_PER_TILE, ROWS_PER_TILE)])

    return sc_kernel


def _tc2_body(h0_ref, p1_ref, pd_ref, w1, b1, w2, b2, h1_ref, st_ref, cnt_ref):
    st = p1_ref[0] + p1_ref[1]
    st_ref[...] = st
    cnt_ref[...] = pd_ref[0] + pd_ref[1]
    pre = h0_ref[...] + st[:, :HID]
    t = _relu(_dot(pre, w1[...]) + b1[...])
    h1_ref[...] = _relu(_dot(t, w2[...]) + b2[...])


def _tc2(h0, part1, partd, w1, b1, w2, b2):
    grid = (N // BLK,)
    wspec = pl.BlockSpec((HID, HID), lambda i: (0, 0))
    bspec = pl.BlockSpec((1, HID), lambda i: (0, 0))
    return pl.pallas_call(
        _tc2_body,
        grid=grid,
        in_specs=[
            pl.BlockSpec((BLK, HID), lambda i: (i, 0)),
            pl.BlockSpec((2, BLK, ROW1), lambda i: (0, i, 0)),
            pl.BlockSpec((2, BLK, DEGW), lambda i: (0, i, 0)),
            wspec, bspec, wspec, bspec,
        ],
        out_specs=[
            pl.BlockSpec((BLK, HID), lambda i: (i, 0)),
            pl.BlockSpec((BLK, ROW1), lambda i: (i, 0)),
            pl.BlockSpec((BLK, DEGW), lambda i: (i, 0)),
        ],
        out_shape=[
            jax.ShapeDtypeStruct((N, HID), jnp.float32),
            jax.ShapeDtypeStruct((N, ROW1), jnp.float32),
            jax.ShapeDtypeStruct((N, DEGW), jnp.float32),
        ],
    )(h0, part1, partd, w1, b1, w2, b2)


def _tc3_body(h0_ref, h1_ref, st_ref, cnt_ref, p2_ref, z_ref, *refs):
    (g1w1, g1b1, g1w2, g1b2,
     hw1, hb1, hw2, hb2, hw3, hb3, dw4, db4,
     sw, sb,
     nw1, nb1, nw2, nb2, nw3, nb3, nw4, nb4,
     l1_ref, scal_ref, h0p_ref) = refs

    h0 = h0_ref[...]
    h1 = h1_ref[...]
    st = st_ref[...]
    s3 = p2_ref[0] + p2_ref[1]

    pre = h1 + s3
    t = _relu(_dot(pre, g1w1[...]) + g1b1[...])
    l1 = _dot(t, g1w2[...]) + g1b2[...]
    l1_ref[...] = l1

    t = _relu(_dot(l1, hw1[...]) + hb1[...])
    t = _relu(_dot(t, hw2[...]) + hb2[...])
    t3 = _dot(t, hw3[...]) + hb3[...]
    deg = _relu(_dot(_relu(t3[:, :HID]), dw4[...]) + db4[...])
    h0p_ref[...] = t3[:, HID:HID + IN_DIM]
    g_mean = t3[:, HID + IN_DIM:2 * HID + IN_DIM]
    g_sigma = t3[:, 2 * HID + IN_DIM:3 * HID + IN_DIM]
    escale = jnp.exp(g_sigma)

    s1 = st[:, :HID]
    s2 = st[:, HID:2 * HID]
    cnt = cnt_ref[...][:, 0:1]
    denom = jnp.maximum(cnt, 1.0)
    mean_n = s1 / denom
    mean_sq = s2 / denom
    std_raw = jnp.sqrt(_relu(mean_sq - mean_n * mean_n) + 1e-5)
    sg = _dot(jnp.concatenate([mean_n, h0, std_raw], axis=1), sw[...]) + sb[...]
    mean_neigh = sg[:, :HID]
    s = sg[:, HID:2 * HID]

    gi = jnp.concatenate([g_mean + escale * z_ref[0],
                          g_mean + escale * z_ref[1]], axis=0)
    t = _relu(_dot(gi, nw1[...]) + nb1[...])
    t = _relu(_dot(t, nw2[...]) + nb2[...])
    t = _relu(_dot(t, nw3[...]) + nb3[...])
    go = _dot(t, nw4[...]) + nb4[...]
    br = h0.shape[0]
    n0 = go[:br]
    n1 = go[br:]
    gen_mean = 0.5 * (n0 + n1)
    u = 0.5 * jnp.abs(n0 - n1)

    ss = jnp.sum(s * s, axis=1, keepdims=True)
    uu = jnp.sum(u * u, axis=1, keepdims=True)
    us = jnp.sum(u * s, axis=1, keepdims=True)
    det_t = 1.0 + ss
    det_g = 1.0 + uu
    trace = HID + ss - (uu + us * us) / det_g
    d = gen_mean - mean_neigh
    zq = (jnp.sum(d * d, axis=1, keepdims=True)
          - jnp.sum(u * d, axis=1, keepdims=True) ** 2 / det_g)
    kl = 0.5 * (jnp.log(det_g / det_t) - HID + trace + zq)
    br = deg.shape[0]
    scal_ref[...] = jnp.concatenate(
        [deg, det_t, det_g, trace, zq, kl, jnp.zeros((br, 2), jnp.float32)],
        axis=1)


def _tc3(h0, h1, st, cnt, part2, z, wlist):
    grid = (N // BLK,)

    def fullspec(a):
        nd = a.ndim
        return pl.BlockSpec(a.shape, lambda i, _nd=nd: (0,) * _nd)

    in_specs = [
        pl.BlockSpec((BLK, HID), lambda i: (i, 0)),
        pl.BlockSpec((BLK, HID), lambda i: (i, 0)),
        pl.BlockSpec((BLK, ROW1), lambda i: (i, 0)),
        pl.BlockSpec((BLK, DEGW), lambda i: (i, 0)),
        pl.BlockSpec((2, BLK, HID), lambda i: (0, i, 0)),
        pl.BlockSpec((2, BLK, HID), lambda i: (0, i, 0)),
    ] + [fullspec(a) for a in wlist]
    return pl.pallas_call(
        _tc3_body,
        grid=grid,
        in_specs=in_specs,
        out_specs=[
            pl.BlockSpec((BLK, HID), lambda i: (i, 0)),
            pl.BlockSpec((BLK, 8), lambda i: (i, 0)),
            pl.BlockSpec((BLK, IN_DIM), lambda i: (i, 0)),
        ],
        out_shape=[
            jax.ShapeDtypeStruct((N, HID), jnp.float32),
            jax.ShapeDtypeStruct((N, 8), jnp.float32),
            jax.ShapeDtypeStruct((N, IN_DIM), jnp.float32),
        ],
    )(h0, h1, st, cnt, part2, z, *wlist)


def _b2(b):
    return b.reshape(1, -1)


def _bdiag(blocks):
    rows = sum(b.shape[0] for b in blocks)
    cols = sum(b.shape[1] for b in blocks)
    out = jnp.zeros((rows, cols), jnp.float32)
    r = c = 0
    for b in blocks:
        out = out.at[r:r + b.shape[0], c:c + b.shape[1]].set(b)
        r += b.shape[0]
        c += b.shape[1]
    return out


def kernel(x, edge_index, params):
    src = edge_index[0].astype(jnp.int32)
    dst = edge_index[1].astype(jnp.int32)
    srcm = jnp.concatenate([src, _PAD_SRC_TAIL]).reshape(KCH, NW, CHUNK)
    dstm = jnp.concatenate([dst, _PAD_DST_TAIL]).reshape(KCH, NW, CHUNK)

    p = params
    lw, lb = p["lin"]
    h0, h0e = _tc1(x, lw, _b2(lb))

    part1, partd = _make_sc_segsum_deg(ROW1)(
        h0e, srcm, dstm, _ZEROS_ROW1, _ZEROS_DEG, _ONES_DEG)

    (g0w1, g0b1), (g0w2, g0b2) = p["gin"][0]
    h1, st, cnt = _tc2(h0, part1, partd, g0w1, _b2(g0b1), g0w2, _b2(g0b2))

    part2 = _make_sc_segsum(HID)(h1, srcm, dstm, _ZEROS_HID)

    dg, ft, mm, ms = p["deg"], p["feat"], p["mlp_mean"], p["mlp_sigma"]
    hw1 = jnp.concatenate([dg[0][0], ft[0][0], mm[0][0], ms[0][0]], axis=1)
    hb1 = jnp.concatenate([dg[0][1], ft[0][1], mm[0][1], ms[0][1]]).reshape(1, -1)
    hw2 = _bdiag([dg[1][0], ft[1][0], mm[1][0], ms[1][0]])
    hb2 = jnp.concatenate([dg[1][1], ft[1][1], mm[1][1], ms[1][1]]).reshape(1, -1)
    hw3 = _bdiag([dg[2][0], ft[2][0], mm[2][0], ms[2][0]])
    hb3 = jnp.concatenate([dg[2][1], ft[2][1], mm[2][1], ms[2][1]]).reshape(1, -1)
    slw, slb = p["sage_l"]
    srw, srb = p["sage_r"]
    stw, stb = p["std_lin"]
    z32 = jnp.zeros((HID, HID), jnp.float32)
    sw = jnp.concatenate([
        jnp.concatenate([slw, z32], axis=1),
        jnp.concatenate([srw, z32], axis=1),
        jnp.concatenate([z32, stw], axis=1)], axis=0)
    sb = jnp.concatenate([slb + srb, stb]).reshape(1, -1)

    (g1w1, g1b1), (g1w2, g1b2) = p["gin"][1]
    wlist = [g1w1, _b2(g1b1), g1w2, _b2(g1b2),
             hw1, hb1, hw2, hb2, hw3, hb3, dg[3][0], _b2(dg[3][1]), sw, sb]
    for (w, b) in p["gen"]:
        wlist += [w, _b2(b)]

    l1, scal, h0p = _tc3(h0, h1, st, cnt, part2, _Z, wlist)

    return (h0, l1, scal[:, 0:1], (h0p,),
            ((scal[:, 1], scal[:, 2], scal[:, 3], scal[:, 4], scal[:, 5]),))
